# Initial kernel scaffold; baseline (speedup 1.0000x reference)
import jax
import jax.numpy as jnp
from jax.experimental import pallas as pl


def kernel(user_table, item_table, W_gc_0, b_gc_0, W_bi_0, b_bi_0, W_gc_1, b_gc_1, W_bi_1, b_bi_1, W_gc_2, b_gc_2, W_bi_2, b_bi_2, vals, rows, cols, user, u_ir, nbr, item, rate):
    raise NotImplementedError("write your pallas kernel here")


# speedup vs baseline: 3.4409x; 3.4409x over previous
import functools

import jax
import jax.numpy as jnp
from jax import lax
from jax.experimental import pallas as pl
from jax.experimental.pallas import tpu as pltpu
from jax.experimental.pallas import tpu_sc as plsc

N_USERS = 25000
N_ITEMS = 25000
N_NODES = N_USERS + N_ITEMS
EDIM = 64
HDIM = 32
NNZ = 800000
B = 4096

NC = 2
NS = 16
SUB = 128
CHUNK_SUBS = 4
CHUNK = SUB * CHUNK_SUBS

N_PAD = 50048
STRIPE = N_PAD // NS
EDGE_ROWS = 6400
NNZ_PAD = EDGE_ROWS * SUB
TILE_ROWS = EDGE_ROWS // (NS)
TILE_CHUNKS = TILE_ROWS // CHUNK_SUBS


def _spmm_body(cols2, rows2, vals2, zeros, ego_l, ego_r, side_l, side_r,
               cols_b, rows_b, vals_b, data_b, acc_sh):
    c = lax.axis_index("c")
    s = lax.axis_index("s")

    pltpu.sync_copy(zeros.at[pl.ds(s * STRIPE, STRIPE)],
                    acc_sh.at[pl.ds(s * STRIPE, STRIPE)])
    plsc.subcore_barrier()

    def run_half(ego_hbm):
        def chunk_body(g, carry):
            row0 = s * TILE_ROWS + g * CHUNK_SUBS
            pltpu.sync_copy(cols2.at[pl.ds(row0, CHUNK_SUBS)], cols_b)
            pltpu.sync_copy(rows2.at[pl.ds(row0, CHUNK_SUBS)], rows_b)
            pltpu.sync_copy(vals2.at[pl.ds(row0, CHUNK_SUBS)], vals_b)
            for j in range(CHUNK_SUBS):
                pltpu.sync_copy(ego_hbm.at[cols_b.at[j]], data_b.at[j])

            def edge_body(e16, carry2):
                for j in range(CHUNK_SUBS):
                    vv = vals_b[j, pl.ds(e16 * 16, 16)]
                    for t in range(16):
                        e = e16 * 16 + t
                        v = vv[t]
                        d0 = data_b[j, e, pl.ds(0, 16)]
                        data_b[j, e, pl.ds(0, 16)] = d0 * v
                        d1 = data_b[j, e, pl.ds(16, 16)]
                        data_b[j, e, pl.ds(16, 16)] = d1 * v
                return carry2

            lax.fori_loop(0, SUB // 16, edge_body, 0)
            for j in range(CHUNK_SUBS):
                pltpu.sync_copy(data_b.at[j], acc_sh.at[rows_b.at[j]],
                                add=True)
            return carry

        lax.fori_loop(0, TILE_CHUNKS, chunk_body, 0)

    @pl.when(c == 0)
    def _():
        run_half(ego_l)

    @pl.when(c == 1)
    def _():
        run_half(ego_r)

    plsc.subcore_barrier()

    @pl.when(c == 0)
    def _():
        pltpu.sync_copy(acc_sh.at[pl.ds(s * STRIPE, STRIPE)],
                        side_l.at[pl.ds(s * STRIPE, STRIPE)])

    @pl.when(c == 1)
    def _():
        pltpu.sync_copy(acc_sh.at[pl.ds(s * STRIPE, STRIPE)],
                        side_r.at[pl.ds(s * STRIPE, STRIPE)])


@jax.jit
def _sc_spmm(cols2, rows2, vals2, zeros, ego_l, ego_r):
    mesh = plsc.VectorSubcoreMesh(core_axis_name="c", subcore_axis_name="s")
    f = pl.kernel(
        _spmm_body,
        out_type=(
            jax.ShapeDtypeStruct((N_PAD, HDIM), jnp.float32),
            jax.ShapeDtypeStruct((N_PAD, HDIM), jnp.float32),
        ),
        mesh=mesh,
        scratch_types=[
            pltpu.VMEM((CHUNK_SUBS, SUB), jnp.int32),
            pltpu.VMEM((CHUNK_SUBS, SUB), jnp.int32),
            pltpu.VMEM((CHUNK_SUBS, SUB), jnp.float32),
            pltpu.VMEM((CHUNK_SUBS, SUB, HDIM), jnp.float32),
            pltpu.VMEM_SHARED((N_PAD, HDIM), jnp.float32),
        ],
        compiler_params=pltpu.CompilerParams(use_tc_tiling_on_sc=False),
    )
    return f(cols2, rows2, vals2, zeros, ego_l, ego_r)


def _dense_body(side_l, side_r, ego_l, ego_r, acc, wg, bg, wb, bb,
                nego_l, nego_r, acc_out):
    side = jnp.concatenate([side_l[...], side_r[...]], axis=1)
    ego = jnp.concatenate([ego_l[...], ego_r[...]], axis=1)
    dn = (((1,), (1,)), ((), ()))
    sum_emb = lax.dot_general(side, wg[...], dn,
                              preferred_element_type=jnp.float32)
    bi_emb = lax.dot_general(ego * side, wb[...], dn,
                             preferred_element_type=jnp.float32)
    h = sum_emb + bi_emb + bg[...] + bb[...]
    ego_new = jnp.where(h >= 0, h, 0.2 * h)
    nrm = jnp.sqrt(jnp.sum(ego_new * ego_new, axis=1, keepdims=True))
    normed = ego_new / jnp.maximum(nrm, 1e-12)
    nego_l[...] = ego_new[:, :HDIM]
    nego_r[...] = ego_new[:, HDIM:]
    acc_out[...] = acc[...] + normed


@jax.jit
def _tc_dense(side_l, side_r, ego_l, ego_r, acc, wg, bg, wb, bb):
    R = 3128
    grid = N_PAD // R
    half = pl.BlockSpec((R, HDIM), lambda i: (i, 0))
    full = pl.BlockSpec((R, EDIM), lambda i: (i, 0))
    wspec = pl.BlockSpec((EDIM, EDIM), lambda i: (0, 0))
    bspec = pl.BlockSpec((1, EDIM), lambda i: (0, 0))
    return pl.pallas_call(
        _dense_body,
        grid=(grid,),
        in_specs=[half, half, half, half, full, wspec, bspec, wspec, bspec],
        out_specs=[half, half, full],
        out_shape=[
            jax.ShapeDtypeStruct((N_PAD, HDIM), jnp.float32),
            jax.ShapeDtypeStruct((N_PAD, HDIM), jnp.float32),
            jax.ShapeDtypeStruct((N_PAD, EDIM), jnp.float32),
        ],
    )(side_l, side_r, ego_l, ego_r, acc, wg, bg, wb, bb)


PER_W = B // (NC * NS)


def _final_body(acc, user_table, item_table, user, item,
                acc_u, acc_i, users_ego, items_ego,
                uidx, iidx, au, ai, tu, ti):
    c = lax.axis_index("c")
    s = lax.axis_index("s")
    wid = s * NC + c
    base = wid * PER_W
    pltpu.sync_copy(user.at[pl.ds(base, PER_W)], uidx)
    pltpu.sync_copy(item.at[pl.ds(base, PER_W)], iidx)
    pltpu.sync_copy(user_table.at[uidx], tu)
    pltpu.sync_copy(item_table.at[iidx], ti)
    pltpu.sync_copy(acc.at[uidx], au)
    for k in range(PER_W // 16):
        iidx[pl.ds(k * 16, 16)] = iidx[pl.ds(k * 16, 16)] + N_USERS
    pltpu.sync_copy(acc.at[iidx], ai)
    pltpu.sync_copy(au, acc_u.at[pl.ds(base, PER_W)])
    pltpu.sync_copy(ai, acc_i.at[pl.ds(base, PER_W)])
    pltpu.sync_copy(tu, users_ego.at[pl.ds(base, PER_W)])
    pltpu.sync_copy(ti, items_ego.at[pl.ds(base, PER_W)])


@jax.jit
def _sc_final(acc, user_table, item_table, user, item):
    mesh = plsc.VectorSubcoreMesh(core_axis_name="c", subcore_axis_name="s")
    f = pl.kernel(
        _final_body,
        out_type=(
            jax.ShapeDtypeStruct((B, EDIM), jnp.float32),
            jax.ShapeDtypeStruct((B, EDIM), jnp.float32),
            jax.ShapeDtypeStruct((B, EDIM), jnp.float32),
            jax.ShapeDtypeStruct((B, EDIM), jnp.float32),
        ),
        mesh=mesh,
        scratch_types=[
            pltpu.VMEM((PER_W,), jnp.int32),
            pltpu.VMEM((PER_W,), jnp.int32),
            pltpu.VMEM((PER_W, EDIM), jnp.float32),
            pltpu.VMEM((PER_W, EDIM), jnp.float32),
            pltpu.VMEM((PER_W, EDIM), jnp.float32),
            pltpu.VMEM((PER_W, EDIM), jnp.float32),
        ],
        compiler_params=pltpu.CompilerParams(use_tc_tiling_on_sc=False),
    )
    return f(acc, user_table, item_table, user, item)


def _pred_body(au, ai, pred):
    pred[...] = jnp.sum(au[...] * ai[...], axis=1) * (1.0 / 16.0)


@jax.jit
def _tc_pred(au, ai):
    return pl.pallas_call(
        _pred_body,
        out_shape=jax.ShapeDtypeStruct((B,), jnp.float32),
    )(au, ai)


def kernel(user_table, item_table, W_gc_0, b_gc_0, W_bi_0, b_bi_0,
           W_gc_1, b_gc_1, W_bi_1, b_bi_1, W_gc_2, b_gc_2, W_bi_2, b_bi_2,
           vals, rows, cols, user, u_ir, nbr, item, rate):
    gc = [(W_gc_0, b_gc_0), (W_gc_1, b_gc_1), (W_gc_2, b_gc_2)]
    bi = [(W_bi_0, b_bi_0), (W_bi_1, b_bi_1), (W_bi_2, b_bi_2)]

    ego = jnp.concatenate([user_table, item_table], axis=0)
    ego_p = jnp.pad(ego, ((0, N_PAD - N_NODES), (0, 0)))
    ego_l = ego_p[:, :HDIM]
    ego_r = ego_p[:, HDIM:]
    acc = ego_p

    epad = NNZ_PAD - NNZ
    cols2 = jnp.pad(cols, (0, epad)).reshape(EDGE_ROWS, SUB)
    rows2 = jnp.pad(rows, (0, epad)).reshape(EDGE_ROWS, SUB)
    vals2 = jnp.pad(vals, (0, epad)).reshape(EDGE_ROWS, SUB)
    zeros = jnp.zeros((N_PAD, HDIM), jnp.float32)

    for l in range(3):
        side_l, side_r = _sc_spmm(cols2, rows2, vals2, zeros, ego_l, ego_r)
        wg = gc[l][0]
        bg = gc[l][1].reshape(1, EDIM)
        wb = bi[l][0]
        bb = bi[l][1].reshape(1, EDIM)
        ego_l, ego_r, acc = _tc_dense(side_l, side_r, ego_l, ego_r, acc,
                                      wg, bg, wb, bb)

    acc_u, acc_i, users_ego, items_ego = _sc_final(acc, user_table,
                                                   item_table, user, item)
    pred = _tc_pred(acc_u, acc_i)
    return (pred, users_ego, items_ego)


# speedup vs baseline: 6.6067x; 1.9200x over previous
import functools

import jax
import jax.numpy as jnp
from jax import lax
from jax.experimental import pallas as pl
from jax.experimental.pallas import tpu as pltpu
from jax.experimental.pallas import tpu_sc as plsc

N_USERS = 25000
N_ITEMS = 25000
N_NODES = N_USERS + N_ITEMS
EDIM = 64
HDIM = 32
NNZ = 800000
B = 4096

NC = 2
NS = 16
SUB = 128
CHUNK_SUBS = 2
CHUNK = SUB * CHUNK_SUBS

N_PAD = 50048
STRIPE = N_PAD // NS
TILE_CHUNKS = 198
TILE_ROWS = TILE_CHUNKS * CHUNK_SUBS
EDGE_ROWS = NS * TILE_ROWS
NNZ_PAD = EDGE_ROWS * SUB


def _spmm_body(cols2, rows2, vals2, zeros, ego_l, ego_r, side_l, side_r,
               cols_b, rows_b, vals_b, data_b,
               sg0, sg1, sg2, ss0, ss1, ss2, sl0, sl1, sl2, acc_sh):
    sem_g = (sg0, sg1, sg2)
    sem_s = (ss0, ss1, ss2)
    sem_ld = (sl0, sl1, sl2)
    c = lax.axis_index("c")
    s = lax.axis_index("s")

    pltpu.sync_copy(zeros.at[pl.ds(s * STRIPE, STRIPE)],
                    acc_sh.at[pl.ds(s * STRIPE, STRIPE)])
    plsc.subcore_barrier()

    def row0_of(g):
        return s * TILE_ROWS + g * CHUNK_SUBS

    def issue_loads(g, p):
        r0 = row0_of(g)
        pltpu.async_copy(cols2.at[pl.ds(r0, CHUNK_SUBS)], cols_b.at[p],
                         sem_ld[p])
        pltpu.async_copy(rows2.at[pl.ds(r0, CHUNK_SUBS)], rows_b.at[p],
                         sem_ld[p])
        pltpu.async_copy(vals2.at[pl.ds(r0, CHUNK_SUBS)], vals_b.at[p],
                         sem_ld[p])

    def wait_loads(g, p):
        r0 = row0_of(g)
        pltpu.make_async_copy(cols2.at[pl.ds(r0, CHUNK_SUBS)], cols_b.at[p],
                              sem_ld[p]).wait()
        pltpu.make_async_copy(rows2.at[pl.ds(r0, CHUNK_SUBS)], rows_b.at[p],
                              sem_ld[p]).wait()
        pltpu.make_async_copy(vals2.at[pl.ds(r0, CHUNK_SUBS)], vals_b.at[p],
                              sem_ld[p]).wait()

    def run_half(ego_hbm, side_hbm):
        def issue_gathers(p):
            for j in range(CHUNK_SUBS):
                pltpu.async_copy(ego_hbm.at[cols_b.at[p, j]],
                                 data_b.at[p, j], sem_g[p])

        def wait_gathers(p):
            for j in range(CHUNK_SUBS):
                pltpu.make_async_copy(ego_hbm.at[cols_b.at[p, j]],
                                      data_b.at[p, j], sem_g[p]).wait()

        def issue_scatters(p):
            for j in range(CHUNK_SUBS):
                pltpu.async_copy(data_b.at[p, j],
                                 acc_sh.at[rows_b.at[p, j]], sem_s[p],
                                 add=True)

        def wait_scatters(p):
            for j in range(CHUNK_SUBS):
                pltpu.make_async_copy(data_b.at[p, j],
                                      acc_sh.at[rows_b.at[p, j]],
                                      sem_s[p]).wait()

        def compute(p):
            def edge_body(e16, carry2):
                for j in range(CHUNK_SUBS):
                    vv = vals_b[p, j, pl.ds(e16 * 16, 16)]
                    for t in range(16):
                        e = e16 * 16 + t
                        v = vv[t]
                        d0 = data_b[p, j, e, pl.ds(0, 16)]
                        data_b[p, j, e, pl.ds(0, 16)] = d0 * v
                        d1 = data_b[p, j, e, pl.ds(16, 16)]
                        data_b[p, j, e, pl.ds(16, 16)] = d1 * v
                return carry2

            lax.fori_loop(0, SUB // 16, edge_body, 0)

        def body(g, p, drain_next_slot, issue_next_gather, issue_far_loads):
            pn = (p + 1) % 3
            if issue_next_gather:
                wait_loads(g + 1, pn)
                if drain_next_slot:
                    wait_scatters(pn)
                issue_gathers(pn)
            wait_gathers(p)
            compute(p)
            issue_scatters(p)
            if issue_far_loads:
                issue_loads(g + 2, (p + 2) % 3)

        issue_loads(0, 0)
        issue_loads(1, 1)
        wait_loads(0, 0)
        issue_gathers(0)
        body(0, 0, False, True, True)
        body(1, 1, False, True, True)

        def triple(t, carry):
            g = 2 + t * 3
            body(g, 2, True, True, True)
            body(g + 1, 0, True, True, True)
            body(g + 2, 1, True, True, True)
            return carry

        lax.fori_loop(0, (TILE_CHUNKS - 6) // 3, triple, 0)
        body(TILE_CHUNKS - 4, 2, True, True, True)
        body(TILE_CHUNKS - 3, 0, True, True, True)
        body(TILE_CHUNKS - 2, 1, True, True, False)
        body(TILE_CHUNKS - 1, 2, False, False, False)
        wait_scatters(0)
        wait_scatters(1)
        wait_scatters(2)

    @pl.when(c == 0)
    def _():
        run_half(ego_l, side_l)

    @pl.when(c == 1)
    def _():
        run_half(ego_r, side_r)

    plsc.subcore_barrier()

    @pl.when(c == 0)
    def _():
        pltpu.sync_copy(acc_sh.at[pl.ds(s * STRIPE, STRIPE)],
                        side_l.at[pl.ds(s * STRIPE, STRIPE)])

    @pl.when(c == 1)
    def _():
        pltpu.sync_copy(acc_sh.at[pl.ds(s * STRIPE, STRIPE)],
                        side_r.at[pl.ds(s * STRIPE, STRIPE)])


@jax.jit
def _sc_spmm(cols2, rows2, vals2, zeros, ego_l, ego_r):
    mesh = plsc.VectorSubcoreMesh(core_axis_name="c", subcore_axis_name="s")
    f = pl.kernel(
        _spmm_body,
        out_type=(
            jax.ShapeDtypeStruct((N_PAD, HDIM), jnp.float32),
            jax.ShapeDtypeStruct((N_PAD, HDIM), jnp.float32),
        ),
        mesh=mesh,
        scratch_types=[
            pltpu.VMEM((3, CHUNK_SUBS, SUB), jnp.int32),
            pltpu.VMEM((3, CHUNK_SUBS, SUB), jnp.int32),
            pltpu.VMEM((3, CHUNK_SUBS, SUB), jnp.float32),
            pltpu.VMEM((3, CHUNK_SUBS, SUB, HDIM), jnp.float32),
            pltpu.SemaphoreType.DMA,
            pltpu.SemaphoreType.DMA,
            pltpu.SemaphoreType.DMA,
            pltpu.SemaphoreType.DMA,
            pltpu.SemaphoreType.DMA,
            pltpu.SemaphoreType.DMA,
            pltpu.SemaphoreType.DMA,
            pltpu.SemaphoreType.DMA,
            pltpu.SemaphoreType.DMA,
            pltpu.VMEM_SHARED((N_PAD, HDIM), jnp.float32),
        ],
        compiler_params=pltpu.CompilerParams(use_tc_tiling_on_sc=False),
    )
    return f(cols2, rows2, vals2, zeros, ego_l, ego_r)


def _dense_body(side_l, side_r, ego_l, ego_r, acc, wcat, bsum,
                nego_l, nego_r, acc_out):
    side = jnp.concatenate([side_l[...], side_r[...]], axis=1)
    ego = jnp.concatenate([ego_l[...], ego_r[...]], axis=1)
    x = jnp.concatenate([side, ego * side], axis=1)
    h = lax.dot_general(x, wcat[...], (((1,), (0,)), ((), ())),
                        preferred_element_type=jnp.float32) + bsum[...]
    ego_new = jnp.where(h >= 0, h, 0.2 * h)
    nrm = jnp.sqrt(jnp.sum(ego_new * ego_new, axis=1, keepdims=True))
    normed = ego_new / jnp.maximum(nrm, 1e-12)
    nego_l[...] = ego_new[:, :HDIM]
    nego_r[...] = ego_new[:, HDIM:]
    acc_out[...] = acc[...] + normed


@jax.jit
def _tc_dense(side_l, side_r, ego_l, ego_r, acc, wcat, bsum):
    R = 3128
    grid = N_PAD // R
    half = pl.BlockSpec((R, HDIM), lambda i: (i, 0))
    full = pl.BlockSpec((R, EDIM), lambda i: (i, 0))
    wspec = pl.BlockSpec((2 * EDIM, EDIM), lambda i: (0, 0))
    bspec = pl.BlockSpec((1, EDIM), lambda i: (0, 0))
    return pl.pallas_call(
        _dense_body,
        grid=(grid,),
        in_specs=[half, half, half, half, full, wspec, bspec],
        out_specs=[half, half, full],
        out_shape=[
            jax.ShapeDtypeStruct((N_PAD, HDIM), jnp.float32),
            jax.ShapeDtypeStruct((N_PAD, HDIM), jnp.float32),
            jax.ShapeDtypeStruct((N_PAD, EDIM), jnp.float32),
        ],
    )(side_l, side_r, ego_l, ego_r, acc, wcat, bsum)


PER_W = B // (NC * NS)


def _final_body(acc, user_table, item_table, user, item,
                acc_u, acc_i, users_ego, items_ego,
                uidx, iidx, au, ai, tu, ti):
    c = lax.axis_index("c")
    s = lax.axis_index("s")
    wid = s * NC + c
    base = wid * PER_W
    pltpu.sync_copy(user.at[pl.ds(base, PER_W)], uidx)
    pltpu.sync_copy(item.at[pl.ds(base, PER_W)], iidx)
    pltpu.sync_copy(user_table.at[uidx], tu)
    pltpu.sync_copy(item_table.at[iidx], ti)
    pltpu.sync_copy(acc.at[uidx], au)
    for k in range(PER_W // 16):
        iidx[pl.ds(k * 16, 16)] = iidx[pl.ds(k * 16, 16)] + N_USERS
    pltpu.sync_copy(acc.at[iidx], ai)
    pltpu.sync_copy(au, acc_u.at[pl.ds(base, PER_W)])
    pltpu.sync_copy(ai, acc_i.at[pl.ds(base, PER_W)])
    pltpu.sync_copy(tu, users_ego.at[pl.ds(base, PER_W)])
    pltpu.sync_copy(ti, items_ego.at[pl.ds(base, PER_W)])


@jax.jit
def _sc_final(acc, user_table, item_table, user, item):
    mesh = plsc.VectorSubcoreMesh(core_axis_name="c", subcore_axis_name="s")
    f = pl.kernel(
        _final_body,
        out_type=(
            jax.ShapeDtypeStruct((B, EDIM), jnp.float32),
            jax.ShapeDtypeStruct((B, EDIM), jnp.float32),
            jax.ShapeDtypeStruct((B, EDIM), jnp.float32),
            jax.ShapeDtypeStruct((B, EDIM), jnp.float32),
        ),
        mesh=mesh,
        scratch_types=[
            pltpu.VMEM((PER_W,), jnp.int32),
            pltpu.VMEM((PER_W,), jnp.int32),
            pltpu.VMEM((PER_W, EDIM), jnp.float32),
            pltpu.VMEM((PER_W, EDIM), jnp.float32),
            pltpu.VMEM((PER_W, EDIM), jnp.float32),
            pltpu.VMEM((PER_W, EDIM), jnp.float32),
        ],
        compiler_params=pltpu.CompilerParams(use_tc_tiling_on_sc=False),
    )
    return f(acc, user_table, item_table, user, item)


def _pred_body(au, ai, pred):
    pred[...] = jnp.sum(au[...] * ai[...], axis=1) * (1.0 / 16.0)


@jax.jit
def _tc_pred(au, ai):
    return pl.pallas_call(
        _pred_body,
        out_shape=jax.ShapeDtypeStruct((B,), jnp.float32),
    )(au, ai)


def kernel(user_table, item_table, W_gc_0, b_gc_0, W_bi_0, b_bi_0,
           W_gc_1, b_gc_1, W_bi_1, b_bi_1, W_gc_2, b_gc_2, W_bi_2, b_bi_2,
           vals, rows, cols, user, u_ir, nbr, item, rate):
    gc = [(W_gc_0, b_gc_0), (W_gc_1, b_gc_1), (W_gc_2, b_gc_2)]
    bi = [(W_bi_0, b_bi_0), (W_bi_1, b_bi_1), (W_bi_2, b_bi_2)]

    ego = jnp.concatenate([user_table, item_table], axis=0)
    ego_p = jnp.pad(ego, ((0, N_PAD - N_NODES), (0, 0)))
    ego_l = ego_p[:, :HDIM]
    ego_r = ego_p[:, HDIM:]
    acc = ego_p

    epad = NNZ_PAD - NNZ
    cols2 = jnp.pad(cols, (0, epad)).reshape(EDGE_ROWS, SUB)
    rows2 = jnp.pad(rows, (0, epad)).reshape(EDGE_ROWS, SUB)
    vals2 = jnp.pad(vals, (0, epad)).reshape(EDGE_ROWS, SUB)
    zeros = jnp.zeros((N_PAD, HDIM), jnp.float32)

    for l in range(3):
        side_l, side_r = _sc_spmm(cols2, rows2, vals2, zeros, ego_l, ego_r)
        wcat = jnp.concatenate([gc[l][0].T, bi[l][0].T], axis=0)
        bsum = (gc[l][1] + bi[l][1]).reshape(1, EDIM)
        ego_l, ego_r, acc = _tc_dense(side_l, side_r, ego_l, ego_r, acc,
                                      wcat, bsum)

    acc_u, acc_i, users_ego, items_ego = _sc_final(acc, user_table,
                                                   item_table, user, item)
    pred = _tc_pred(acc_u, acc_i)
    return (pred, users_ego, items_ego)


# speedup vs baseline: 7.0023x; 1.0599x over previous
import functools

import jax
import jax.numpy as jnp
from jax import lax
from jax.experimental import pallas as pl
from jax.experimental.pallas import tpu as pltpu
from jax.experimental.pallas import tpu_sc as plsc

N_USERS = 25000
N_ITEMS = 25000
N_NODES = N_USERS + N_ITEMS
EDIM = 64
HDIM = 32
NNZ = 800000
B = 4096

NC = 2
NS = 16
SUB = 128
CHUNK_SUBS = 2
CHUNK = SUB * CHUNK_SUBS

N_PAD = 50048
STRIPE = N_PAD // NS
TILE_CHUNKS = 198
TILE_ROWS = TILE_CHUNKS * CHUNK_SUBS
EDGE_ROWS = NS * TILE_ROWS
NNZ_PAD = EDGE_ROWS * SUB


def _spmm_body(meta_c, vals_c, zeros, ego_l, ego_r, side_l, side_r,
               meta_b, vals_b, data_b,
               sg0, sg1, sg2, ss0, ss1, ss2, sl0, sl1, sl2, acc_sh):
    sem_g = (sg0, sg1, sg2)
    sem_s = (ss0, ss1, ss2)
    sem_ld = (sl0, sl1, sl2)
    c = lax.axis_index("c")
    s = lax.axis_index("s")

    pltpu.sync_copy(zeros.at[pl.ds(s * STRIPE, STRIPE)],
                    acc_sh.at[pl.ds(s * STRIPE, STRIPE)])
    plsc.subcore_barrier()

    def chunk_of(g):
        return s * TILE_CHUNKS + g

    def issue_loads(g, p):
        pltpu.async_copy(meta_c.at[chunk_of(g)], meta_b.at[p], sem_ld[p])
        pltpu.async_copy(vals_c.at[chunk_of(g)], vals_b.at[p], sem_ld[p])

    def wait_loads(g, p):
        pltpu.make_async_copy(meta_c.at[chunk_of(g)], meta_b.at[p],
                              sem_ld[p]).wait()
        pltpu.make_async_copy(vals_c.at[chunk_of(g)], vals_b.at[p],
                              sem_ld[p]).wait()

    def run_half(ego_hbm, side_hbm):
        def issue_gathers(p):
            pltpu.async_copy(ego_hbm.at[meta_b.at[p, 0]], data_b.at[p],
                             sem_g[p])

        def wait_gathers(p):
            pltpu.make_async_copy(ego_hbm.at[meta_b.at[p, 0]],
                                  data_b.at[p], sem_g[p]).wait()

        def issue_scatters(p):
            pltpu.async_copy(data_b.at[p], acc_sh.at[meta_b.at[p, 1]],
                             sem_s[p], add=True)

        def wait_scatters(p):
            pltpu.make_async_copy(data_b.at[p], acc_sh.at[meta_b.at[p, 1]],
                                  sem_s[p]).wait()

        def compute(p):
            def edge_body(e16, carry2):
                vv = vals_b[p, pl.ds(e16 * 16, 16)]
                for t in range(16):
                    e = e16 * 16 + t
                    v = vv[t]
                    d0 = data_b[p, e, pl.ds(0, 16)]
                    data_b[p, e, pl.ds(0, 16)] = d0 * v
                    d1 = data_b[p, e, pl.ds(16, 16)]
                    data_b[p, e, pl.ds(16, 16)] = d1 * v
                return carry2

            lax.fori_loop(0, CHUNK // 16, edge_body, 0)

        def body(g, p, drain_next_slot, issue_next_gather, issue_far_loads):
            pn = (p + 1) % 3
            if issue_next_gather:
                wait_loads(g + 1, pn)
                if drain_next_slot:
                    wait_scatters(pn)
                issue_gathers(pn)
            wait_gathers(p)
            compute(p)
            issue_scatters(p)
            if issue_far_loads:
                issue_loads(g + 2, (p + 2) % 3)

        issue_loads(0, 0)
        issue_loads(1, 1)
        wait_loads(0, 0)
        issue_gathers(0)
        body(0, 0, False, True, True)
        body(1, 1, False, True, True)

        def triple(t, carry):
            g = 2 + t * 3
            body(g, 2, True, True, True)
            body(g + 1, 0, True, True, True)
            body(g + 2, 1, True, True, True)
            return carry

        lax.fori_loop(0, (TILE_CHUNKS - 6) // 3, triple, 0)
        body(TILE_CHUNKS - 4, 2, True, True, True)
        body(TILE_CHUNKS - 3, 0, True, True, True)
        body(TILE_CHUNKS - 2, 1, True, True, False)
        body(TILE_CHUNKS - 1, 2, False, False, False)
        wait_scatters(0)
        wait_scatters(1)
        wait_scatters(2)

    @pl.when(c == 0)
    def _():
        run_half(ego_l, side_l)

    @pl.when(c == 1)
    def _():
        run_half(ego_r, side_r)

    plsc.subcore_barrier()

    @pl.when(c == 0)
    def _():
        pltpu.sync_copy(acc_sh.at[pl.ds(s * STRIPE, STRIPE)],
                        side_l.at[pl.ds(s * STRIPE, STRIPE)])

    @pl.when(c == 1)
    def _():
        pltpu.sync_copy(acc_sh.at[pl.ds(s * STRIPE, STRIPE)],
                        side_r.at[pl.ds(s * STRIPE, STRIPE)])


@jax.jit
def _sc_spmm(meta_c, vals_c, zeros, ego_l, ego_r):
    mesh = plsc.VectorSubcoreMesh(core_axis_name="c", subcore_axis_name="s")
    f = pl.kernel(
        _spmm_body,
        out_type=(
            jax.ShapeDtypeStruct((N_PAD, HDIM), jnp.float32),
            jax.ShapeDtypeStruct((N_PAD, HDIM), jnp.float32),
        ),
        mesh=mesh,
        scratch_types=[
            pltpu.VMEM((3, 2, CHUNK_SUBS * SUB), jnp.int32),
            pltpu.VMEM((3, CHUNK_SUBS * SUB), jnp.float32),
            pltpu.VMEM((3, CHUNK_SUBS * SUB, HDIM), jnp.float32),
            pltpu.SemaphoreType.DMA,
            pltpu.SemaphoreType.DMA,
            pltpu.SemaphoreType.DMA,
            pltpu.SemaphoreType.DMA,
            pltpu.SemaphoreType.DMA,
            pltpu.SemaphoreType.DMA,
            pltpu.SemaphoreType.DMA,
            pltpu.SemaphoreType.DMA,
            pltpu.SemaphoreType.DMA,
            pltpu.VMEM_SHARED((N_PAD, HDIM), jnp.float32),
        ],
        compiler_params=pltpu.CompilerParams(use_tc_tiling_on_sc=False),
    )
    return f(meta_c, vals_c, zeros, ego_l, ego_r)


def _dense_body(side_l, side_r, ego_l, ego_r, acc, wcat, bsum,
                nego_l, nego_r, acc_out):
    side = jnp.concatenate([side_l[...], side_r[...]], axis=1)
    ego = jnp.concatenate([ego_l[...], ego_r[...]], axis=1)
    x = jnp.concatenate([side, ego * side], axis=1)
    h = lax.dot_general(x, wcat[...], (((1,), (0,)), ((), ())),
                        preferred_element_type=jnp.float32) + bsum[...]
    ego_new = jnp.where(h >= 0, h, 0.2 * h)
    nrm = jnp.sqrt(jnp.sum(ego_new * ego_new, axis=1, keepdims=True))
    normed = ego_new / jnp.maximum(nrm, 1e-12)
    nego_l[...] = ego_new[:, :HDIM]
    nego_r[...] = ego_new[:, HDIM:]
    acc_out[...] = acc[...] + normed


@jax.jit
def _tc_dense(side_l, side_r, ego_l, ego_r, acc, wcat, bsum):
    R = 3128
    grid = N_PAD // R
    half = pl.BlockSpec((R, HDIM), lambda i: (i, 0))
    full = pl.BlockSpec((R, EDIM), lambda i: (i, 0))
    wspec = pl.BlockSpec((2 * EDIM, EDIM), lambda i: (0, 0))
    bspec = pl.BlockSpec((1, EDIM), lambda i: (0, 0))
    return pl.pallas_call(
        _dense_body,
        grid=(grid,),
        in_specs=[half, half, half, half, full, wspec, bspec],
        out_specs=[half, half, full],
        out_shape=[
            jax.ShapeDtypeStruct((N_PAD, HDIM), jnp.float32),
            jax.ShapeDtypeStruct((N_PAD, HDIM), jnp.float32),
            jax.ShapeDtypeStruct((N_PAD, EDIM), jnp.float32),
        ],
    )(side_l, side_r, ego_l, ego_r, acc, wcat, bsum)


PER_W = B // (NC * NS)


def _final_body(acc, user_table, item_table, user, item,
                acc_u, acc_i, users_ego, items_ego,
                uidx, iidx, au, ai, tu, ti):
    c = lax.axis_index("c")
    s = lax.axis_index("s")
    wid = s * NC + c
    base = wid * PER_W
    pltpu.sync_copy(user.at[pl.ds(base, PER_W)], uidx)
    pltpu.sync_copy(item.at[pl.ds(base, PER_W)], iidx)
    pltpu.sync_copy(user_table.at[uidx], tu)
    pltpu.sync_copy(item_table.at[iidx], ti)
    pltpu.sync_copy(acc.at[uidx], au)
    for k in range(PER_W // 16):
        iidx[pl.ds(k * 16, 16)] = iidx[pl.ds(k * 16, 16)] + N_USERS
    pltpu.sync_copy(acc.at[iidx], ai)
    pltpu.sync_copy(au, acc_u.at[pl.ds(base, PER_W)])
    pltpu.sync_copy(ai, acc_i.at[pl.ds(base, PER_W)])
    pltpu.sync_copy(tu, users_ego.at[pl.ds(base, PER_W)])
    pltpu.sync_copy(ti, items_ego.at[pl.ds(base, PER_W)])


@jax.jit
def _sc_final(acc, user_table, item_table, user, item):
    mesh = plsc.VectorSubcoreMesh(core_axis_name="c", subcore_axis_name="s")
    f = pl.kernel(
        _final_body,
        out_type=(
            jax.ShapeDtypeStruct((B, EDIM), jnp.float32),
            jax.ShapeDtypeStruct((B, EDIM), jnp.float32),
            jax.ShapeDtypeStruct((B, EDIM), jnp.float32),
            jax.ShapeDtypeStruct((B, EDIM), jnp.float32),
        ),
        mesh=mesh,
        scratch_types=[
            pltpu.VMEM((PER_W,), jnp.int32),
            pltpu.VMEM((PER_W,), jnp.int32),
            pltpu.VMEM((PER_W, EDIM), jnp.float32),
            pltpu.VMEM((PER_W, EDIM), jnp.float32),
            pltpu.VMEM((PER_W, EDIM), jnp.float32),
            pltpu.VMEM((PER_W, EDIM), jnp.float32),
        ],
        compiler_params=pltpu.CompilerParams(use_tc_tiling_on_sc=False),
    )
    return f(acc, user_table, item_table, user, item)


def _pred_body(au, ai, pred):
    pred[...] = jnp.sum(au[...] * ai[...], axis=1) * (1.0 / 16.0)


@jax.jit
def _tc_pred(au, ai):
    return pl.pallas_call(
        _pred_body,
        out_shape=jax.ShapeDtypeStruct((B,), jnp.float32),
    )(au, ai)


def kernel(user_table, item_table, W_gc_0, b_gc_0, W_bi_0, b_bi_0,
           W_gc_1, b_gc_1, W_bi_1, b_bi_1, W_gc_2, b_gc_2, W_bi_2, b_bi_2,
           vals, rows, cols, user, u_ir, nbr, item, rate):
    gc = [(W_gc_0, b_gc_0), (W_gc_1, b_gc_1), (W_gc_2, b_gc_2)]
    bi = [(W_bi_0, b_bi_0), (W_bi_1, b_bi_1), (W_bi_2, b_bi_2)]

    ego = jnp.concatenate([user_table, item_table], axis=0)
    ego_p = jnp.pad(ego, ((0, N_PAD - N_NODES), (0, 0)))
    ego_l = ego_p[:, :HDIM]
    ego_r = ego_p[:, HDIM:]
    acc = ego_p

    epad = NNZ_PAD - NNZ
    cshape = (NS * TILE_CHUNKS, CHUNK_SUBS * SUB)
    cols2 = jnp.pad(cols, (0, epad)).reshape(cshape)
    rows2 = jnp.pad(rows, (0, epad)).reshape(cshape)
    vals2 = jnp.pad(vals, (0, epad)).reshape(cshape)
    meta_c = jnp.stack([cols2, rows2], axis=1)
    zeros = jnp.zeros((N_PAD, HDIM), jnp.float32)

    for l in range(3):
        side_l, side_r = _sc_spmm(meta_c, vals2, zeros, ego_l, ego_r)
        wcat = jnp.concatenate([gc[l][0].T, bi[l][0].T], axis=0)
        bsum = (gc[l][1] + bi[l][1]).reshape(1, EDIM)
        ego_l, ego_r, acc = _tc_dense(side_l, side_r, ego_l, ego_r, acc,
                                      wcat, bsum)

    acc_u, acc_i, users_ego, items_ego = _sc_final(acc, user_table,
                                                   item_table, user, item)
    pred = _tc_pred(acc_u, acc_i)
    return (pred, users_ego, items_ego)


# speedup vs baseline: 8.6758x; 1.2390x over previous
import functools

import jax
import jax.numpy as jnp
from jax import lax
from jax.experimental import pallas as pl
from jax.experimental.pallas import tpu as pltpu
from jax.experimental.pallas import tpu_sc as plsc

N_USERS = 25000
N_ITEMS = 25000
N_NODES = N_USERS + N_ITEMS
EDIM = 64
HDIM = 32
NNZ = 800000
B = 4096

NC = 2
NS = 16
SUB = 128
CHUNK_SUBS = 2
CHUNK = SUB * CHUNK_SUBS

N_PAD = 50048
STRIPE = N_PAD // NS
TILE_CHUNKS = 198
TILE_ROWS = TILE_CHUNKS * CHUNK_SUBS
EDGE_ROWS = NS * TILE_ROWS
NNZ_PAD = EDGE_ROWS * SUB


def _spmm_body(meta_c, vals_c, zeros, ego_l, ego_r, side_l, side_r,
               meta_b, vals_b, data_b,
               sg0, sg1, sg2, ss0, ss1, ss2, sl0, sl1, sl2, acc_sh):
    sem_g = (sg0, sg1, sg2)
    sem_s = (ss0, ss1, ss2)
    sem_ld = (sl0, sl1, sl2)
    c = lax.axis_index("c")
    s = lax.axis_index("s")

    pltpu.sync_copy(zeros.at[pl.ds(s * STRIPE, STRIPE)],
                    acc_sh.at[pl.ds(s * STRIPE, STRIPE)])
    plsc.subcore_barrier()

    def chunk_of(g):
        return s * TILE_CHUNKS + g

    def issue_loads(g, p):
        pltpu.async_copy(meta_c.at[chunk_of(g)], meta_b.at[p], sem_ld[p])
        pltpu.async_copy(vals_c.at[chunk_of(g)], vals_b.at[p], sem_ld[p])

    def wait_loads(g, p):
        pltpu.make_async_copy(meta_c.at[chunk_of(g)], meta_b.at[p],
                              sem_ld[p]).wait()
        pltpu.make_async_copy(vals_c.at[chunk_of(g)], vals_b.at[p],
                              sem_ld[p]).wait()

    def run_half(ego_hbm, side_hbm):
        def issue_gathers(p):
            pltpu.async_copy(ego_hbm.at[meta_b.at[p, 0]], data_b.at[p],
                             sem_g[p])

        def wait_gathers(p):
            pltpu.make_async_copy(ego_hbm.at[meta_b.at[p, 0]],
                                  data_b.at[p], sem_g[p]).wait()

        def issue_scatters(p):
            pltpu.async_copy(data_b.at[p], acc_sh.at[meta_b.at[p, 1]],
                             sem_s[p], add=True)

        def wait_scatters(p):
            pltpu.make_async_copy(data_b.at[p], acc_sh.at[meta_b.at[p, 1]],
                                  sem_s[p]).wait()

        def compute(p):
            def edge_body(e16, carry2):
                vv = vals_b[p, pl.ds(e16 * 16, 16)]
                for t in range(16):
                    e = e16 * 16 + t
                    v = vv[t]
                    d0 = data_b[p, e, pl.ds(0, 16)]
                    data_b[p, e, pl.ds(0, 16)] = d0 * v
                    d1 = data_b[p, e, pl.ds(16, 16)]
                    data_b[p, e, pl.ds(16, 16)] = d1 * v
                return carry2

            lax.fori_loop(0, CHUNK // 16, edge_body, 0)

        def body(g, p, drain_next_slot, issue_next_gather, issue_far_loads):
            pn = (p + 1) % 3
            if issue_next_gather:
                wait_loads(g + 1, pn)
                if drain_next_slot:
                    wait_scatters(pn)
                issue_gathers(pn)
            wait_gathers(p)
            compute(p)
            issue_scatters(p)
            if issue_far_loads:
                issue_loads(g + 2, (p + 2) % 3)

        issue_loads(0, 0)
        issue_loads(1, 1)
        wait_loads(0, 0)
        issue_gathers(0)
        body(0, 0, False, True, True)
        body(1, 1, False, True, True)

        def triple(t, carry):
            g = 2 + t * 3
            body(g, 2, True, True, True)
            body(g + 1, 0, True, True, True)
            body(g + 2, 1, True, True, True)
            return carry

        lax.fori_loop(0, (TILE_CHUNKS - 6) // 3, triple, 0)
        body(TILE_CHUNKS - 4, 2, True, True, True)
        body(TILE_CHUNKS - 3, 0, True, True, True)
        body(TILE_CHUNKS - 2, 1, True, True, False)
        body(TILE_CHUNKS - 1, 2, False, False, False)
        wait_scatters(0)
        wait_scatters(1)
        wait_scatters(2)

    @pl.when(c == 0)
    def _():
        run_half(ego_l, side_l)

    @pl.when(c == 1)
    def _():
        run_half(ego_r, side_r)

    plsc.subcore_barrier()

    @pl.when(c == 0)
    def _():
        pltpu.sync_copy(acc_sh.at[pl.ds(s * STRIPE, STRIPE)],
                        side_l.at[pl.ds(s * STRIPE, STRIPE)])

    @pl.when(c == 1)
    def _():
        pltpu.sync_copy(acc_sh.at[pl.ds(s * STRIPE, STRIPE)],
                        side_r.at[pl.ds(s * STRIPE, STRIPE)])


@jax.jit
def _sc_spmm(meta_c, vals_c, zeros, ego_l, ego_r):
    mesh = plsc.VectorSubcoreMesh(core_axis_name="c", subcore_axis_name="s")
    f = pl.kernel(
        _spmm_body,
        out_type=(
            jax.ShapeDtypeStruct((N_PAD, HDIM), jnp.float32),
            jax.ShapeDtypeStruct((N_PAD, HDIM), jnp.float32),
        ),
        mesh=mesh,
        scratch_types=[
            pltpu.VMEM((3, 2, CHUNK_SUBS * SUB), jnp.int32),
            pltpu.VMEM((3, CHUNK_SUBS * SUB), jnp.float32),
            pltpu.VMEM((3, CHUNK_SUBS * SUB, HDIM), jnp.float32),
            pltpu.SemaphoreType.DMA,
            pltpu.SemaphoreType.DMA,
            pltpu.SemaphoreType.DMA,
            pltpu.SemaphoreType.DMA,
            pltpu.SemaphoreType.DMA,
            pltpu.SemaphoreType.DMA,
            pltpu.SemaphoreType.DMA,
            pltpu.SemaphoreType.DMA,
            pltpu.SemaphoreType.DMA,
            pltpu.VMEM_SHARED((N_PAD, HDIM), jnp.float32),
        ],
        compiler_params=pltpu.CompilerParams(use_tc_tiling_on_sc=False),
    )
    return f(meta_c, vals_c, zeros, ego_l, ego_r)


N_Q = N_PAD // 4


def _dense_body(sl, sr, el, er, al, ar, wl, wr, b4l, b4r, s32, s32t,
                nl, nr, aol, aor):
    x = jnp.concatenate([sl[...], sr[...]], axis=1)
    e = jnp.concatenate([el[...], er[...]], axis=1)
    xx = jnp.concatenate([x, e * x], axis=1)
    dn = (((1,), (0,)), ((), ()))
    hl = lax.dot_general(xx, wl[...], dn,
                         preferred_element_type=jnp.float32) + b4l[...]
    hr = lax.dot_general(xx, wr[...], dn,
                         preferred_element_type=jnp.float32) + b4r[...]
    gl = jnp.where(hl >= 0, hl, 0.2 * hl)
    gr = jnp.where(hr >= 0, hr, 0.2 * hr)
    n2 = (lax.dot_general(gl * gl, s32[...], dn,
                          preferred_element_type=jnp.float32) +
          lax.dot_general(gr * gr, s32[...], dn,
                          preferred_element_type=jnp.float32))
    inv = 1.0 / jnp.maximum(jnp.sqrt(n2), 1e-12)
    inv128 = lax.dot_general(inv, s32t[...], dn,
                             preferred_element_type=jnp.float32)
    nl[...] = gl
    nr[...] = gr
    aol[...] = al[...] + gl * inv128
    aor[...] = ar[...] + gr * inv128


@jax.jit
def _tc_dense(sl, sr, el, er, al, ar, wl, wr, b4l, b4r, s32, s32t):
    R = 3128
    grid = N_Q // R
    blk = pl.BlockSpec((R, 128), lambda i: (i, 0))
    wspec = pl.BlockSpec((512, 128), lambda i: (0, 0))
    bspec = pl.BlockSpec((1, 128), lambda i: (0, 0))
    s32spec = pl.BlockSpec((128, 4), lambda i: (0, 0))
    s32tspec = pl.BlockSpec((4, 128), lambda i: (0, 0))
    out = jax.ShapeDtypeStruct((N_Q, 128), jnp.float32)
    return pl.pallas_call(
        _dense_body,
        grid=(grid,),
        in_specs=[blk, blk, blk, blk, blk, blk, wspec, wspec, bspec, bspec,
                  s32spec, s32tspec],
        out_specs=[blk, blk, blk, blk],
        out_shape=[out, out, out, out],
    )(sl, sr, el, er, al, ar, wl, wr, b4l, b4r, s32, s32t)


PER_W = B // (NC * NS)


def _lane_perm(v, mask_xor):
    idx = (jnp.arange(16, dtype=jnp.int32) ^ mask_xor)[:, None]
    dn = lax.GatherDimensionNumbers(offset_dims=(),
                                    collapsed_slice_dims=(0,),
                                    start_index_map=(0,))
    return lax.gather(v, idx, dn, slice_sizes=(1,),
                      mode=lax.GatherScatterMode.PROMISE_IN_BOUNDS)


def _final_body(acc_l, acc_r, user_table, item_table, user, item,
                pred, users_ego, items_ego,
                uidx, iidx, aul, aur, ail, air, tu, ti, pred_b):
    c = lax.axis_index("c")
    s = lax.axis_index("s")
    wid = s * NC + c
    base = wid * PER_W
    pltpu.sync_copy(user.at[pl.ds(base, PER_W)], uidx)
    pltpu.sync_copy(item.at[pl.ds(base, PER_W)], iidx)
    pltpu.sync_copy(user_table.at[uidx], tu)
    pltpu.sync_copy(item_table.at[iidx], ti)
    pltpu.sync_copy(acc_l.at[uidx], aul)
    pltpu.sync_copy(acc_r.at[uidx], aur)
    for k in range(PER_W // 16):
        iidx[pl.ds(k * 16, 16)] = iidx[pl.ds(k * 16, 16)] + N_USERS
    pltpu.sync_copy(acc_l.at[iidx], ail)
    pltpu.sync_copy(acc_r.at[iidx], air)
    pltpu.sync_copy(tu, users_ego.at[pl.ds(base, PER_W)])
    pltpu.sync_copy(ti, items_ego.at[pl.ds(base, PER_W)])

    lanes = jnp.arange(16, dtype=jnp.int32)

    def group_body(p16, carry):
        dots = jnp.zeros((16,), jnp.float32)
        for t in range(16):
            p = p16 * 16 + t
            sv = aul[p, pl.ds(0, 16)] * ail[p, pl.ds(0, 16)]
            sv = sv + aul[p, pl.ds(16, 16)] * ail[p, pl.ds(16, 16)]
            sv = sv + aur[p, pl.ds(0, 16)] * air[p, pl.ds(0, 16)]
            sv = sv + aur[p, pl.ds(16, 16)] * air[p, pl.ds(16, 16)]
            for k in (1, 2, 4, 8):
                sv = sv + _lane_perm(sv, k)
            dots = jnp.where(lanes == t, sv, dots)
        pred_b[pl.ds(p16 * 16, 16)] = dots * (1.0 / 16.0)
        return carry

    lax.fori_loop(0, PER_W // 16, group_body, 0)
    pltpu.sync_copy(pred_b, pred.at[pl.ds(base, PER_W)])


@jax.jit
def _sc_final(acc_l, acc_r, user_table, item_table, user, item):
    mesh = plsc.VectorSubcoreMesh(core_axis_name="c", subcore_axis_name="s")
    f = pl.kernel(
        _final_body,
        out_type=(
            jax.ShapeDtypeStruct((B,), jnp.float32),
            jax.ShapeDtypeStruct((B, EDIM), jnp.float32),
            jax.ShapeDtypeStruct((B, EDIM), jnp.float32),
        ),
        mesh=mesh,
        scratch_types=[
            pltpu.VMEM((PER_W,), jnp.int32),
            pltpu.VMEM((PER_W,), jnp.int32),
            pltpu.VMEM((PER_W, HDIM), jnp.float32),
            pltpu.VMEM((PER_W, HDIM), jnp.float32),
            pltpu.VMEM((PER_W, HDIM), jnp.float32),
            pltpu.VMEM((PER_W, HDIM), jnp.float32),
            pltpu.VMEM((PER_W, EDIM), jnp.float32),
            pltpu.VMEM((PER_W, EDIM), jnp.float32),
            pltpu.VMEM((PER_W,), jnp.float32),
        ],
        compiler_params=pltpu.CompilerParams(use_tc_tiling_on_sc=False),
    )
    return f(acc_l, acc_r, user_table, item_table, user, item)


def kernel(user_table, item_table, W_gc_0, b_gc_0, W_bi_0, b_bi_0,
           W_gc_1, b_gc_1, W_bi_1, b_bi_1, W_gc_2, b_gc_2, W_bi_2, b_bi_2,
           vals, rows, cols, user, u_ir, nbr, item, rate):
    gc = [(W_gc_0, b_gc_0), (W_gc_1, b_gc_1), (W_gc_2, b_gc_2)]
    bi = [(W_bi_0, b_bi_0), (W_bi_1, b_bi_1), (W_bi_2, b_bi_2)]

    ego = jnp.concatenate([user_table, item_table], axis=0)
    ego_p = jnp.pad(ego, ((0, N_PAD - N_NODES), (0, 0)))
    ego_l = ego_p[:, :HDIM]
    ego_r = ego_p[:, HDIM:]
    el_p = ego_l.reshape(N_Q, 128)
    er_p = ego_r.reshape(N_Q, 128)
    al_p = el_p
    ar_p = er_p

    epad = NNZ_PAD - NNZ
    cshape = (NS * TILE_CHUNKS, CHUNK_SUBS * SUB)
    cols2 = jnp.pad(cols, (0, epad)).reshape(cshape)
    rows2 = jnp.pad(rows, (0, epad)).reshape(cshape)
    vals2 = jnp.pad(vals, (0, epad)).reshape(cshape)
    meta_c = jnp.stack([cols2, rows2], axis=1)
    zeros = jnp.zeros((N_PAD, HDIM), jnp.float32)

    eye4 = jnp.eye(4, dtype=jnp.float32)
    s32 = jnp.kron(eye4, jnp.ones((HDIM, 1), jnp.float32))
    s32t = jnp.kron(eye4, jnp.ones((1, HDIM), jnp.float32))

    for l in range(3):
        side_l, side_r = _sc_spmm(meta_c, vals2, zeros, ego_l, ego_r)
        a = gc[l][0].T
        b = bi[l][0].T
        wl = jnp.concatenate([
            jnp.kron(eye4, a[:HDIM, :HDIM]),
            jnp.kron(eye4, a[HDIM:, :HDIM]),
            jnp.kron(eye4, b[:HDIM, :HDIM]),
            jnp.kron(eye4, b[HDIM:, :HDIM]),
        ], axis=0)
        wr = jnp.concatenate([
            jnp.kron(eye4, a[:HDIM, HDIM:]),
            jnp.kron(eye4, a[HDIM:, HDIM:]),
            jnp.kron(eye4, b[:HDIM, HDIM:]),
            jnp.kron(eye4, b[HDIM:, HDIM:]),
        ], axis=0)
        bsum = gc[l][1] + bi[l][1]
        b4l = jnp.tile(bsum[:HDIM], 4).reshape(1, 128)
        b4r = jnp.tile(bsum[HDIM:], 4).reshape(1, 128)
        el_p, er_p, al_p, ar_p = _tc_dense(
            side_l.reshape(N_Q, 128), side_r.reshape(N_Q, 128),
            el_p, er_p, al_p, ar_p, wl, wr, b4l, b4r, s32, s32t)
        ego_l = el_p.reshape(N_PAD, HDIM)
        ego_r = er_p.reshape(N_PAD, HDIM)

    pred, users_ego, items_ego = _sc_final(al_p.reshape(N_PAD, HDIM),
                                           ar_p.reshape(N_PAD, HDIM),
                                           user_table, item_table,
                                           user, item)
    return (pred, users_ego, items_ego)


# speedup vs baseline: 8.9094x; 1.0269x over previous
import functools

import jax
import jax.numpy as jnp
from jax import lax
from jax.experimental import pallas as pl
from jax.experimental.pallas import tpu as pltpu
from jax.experimental.pallas import tpu_sc as plsc

N_USERS = 25000
N_ITEMS = 25000
N_NODES = N_USERS + N_ITEMS
EDIM = 64
HDIM = 32
NNZ = 800000
B = 4096

NC = 2
NS = 16
CHUNK = 192
TILE_CHUNKS = 264
NNZ_PAD = NS * TILE_CHUNKS * CHUNK

N_PAD = 50048
STRIPE = N_PAD // NS
DRING = 4
MRING = 6


def _spmm_body(meta_c, vals_c, zeros, ego_l, ego_r, side_l, side_r,
               meta_b, vals_b, data_b, sem_g, sem_s, sem_ld, acc_sh):
    c = lax.axis_index("c")
    s = lax.axis_index("s")

    pltpu.sync_copy(zeros.at[pl.ds(s * STRIPE, STRIPE)],
                    acc_sh.at[pl.ds(s * STRIPE, STRIPE)])
    plsc.subcore_barrier()

    def chunk_of(g):
        return s * TILE_CHUNKS + g

    def issue_loads(g, m):
        pltpu.async_copy(meta_c.at[chunk_of(g)], meta_b.at[m], sem_ld.at[m])
        pltpu.async_copy(vals_c.at[chunk_of(g)], vals_b.at[m], sem_ld.at[m])

    def wait_loads(g, m):
        pltpu.make_async_copy(meta_c.at[chunk_of(g)], meta_b.at[m],
                              sem_ld.at[m]).wait()
        pltpu.make_async_copy(vals_c.at[chunk_of(g)], vals_b.at[m],
                              sem_ld.at[m]).wait()

    def run_half(ego_hbm, side_hbm):
        def issue_gather(p, m):
            pltpu.async_copy(ego_hbm.at[meta_b.at[m, 0]], data_b.at[p],
                             sem_g.at[p])

        def wait_gather(p, m):
            pltpu.make_async_copy(ego_hbm.at[meta_b.at[m, 0]],
                                  data_b.at[p], sem_g.at[p]).wait()

        def issue_scatter(p, m):
            pltpu.async_copy(data_b.at[p], acc_sh.at[meta_b.at[m, 1]],
                             sem_s.at[p], add=True)

        def wait_scatter(p, m):
            pltpu.make_async_copy(data_b.at[p], acc_sh.at[meta_b.at[m, 1]],
                                  sem_s.at[p]).wait()

        def compute(p, m):
            def edge_body(e16, carry2):
                vv = vals_b[m, pl.ds(e16 * 16, 16)]
                for t in range(16):
                    e = e16 * 16 + t
                    v = vv[t]
                    d0 = data_b[p, e, pl.ds(0, 16)]
                    data_b[p, e, pl.ds(0, 16)] = d0 * v
                    d1 = data_b[p, e, pl.ds(16, 16)]
                    data_b[p, e, pl.ds(16, 16)] = d1 * v
                return carry2

            lax.fori_loop(0, CHUNK // 16, edge_body, 0)

        for m in range(4):
            issue_loads(m, m)
        wait_loads(0, 0)
        issue_gather(0, 0)
        wait_loads(1, 1)
        issue_gather(1, 1)

        def body(g, carry):
            p4 = lax.rem(g, DRING)
            p6 = lax.rem(g, MRING)
            p4n = lax.rem(g + 2, DRING)
            p6n = lax.rem(g + 2, MRING)

            @pl.when(jnp.logical_and(g >= 2, g + 2 < TILE_CHUNKS))
            def _():
                wait_scatter(p4n, p6n)

            @pl.when(g + 2 < TILE_CHUNKS)
            def _():
                wait_loads(g + 2, p6n)
                issue_gather(p4n, p6n)

            @pl.when(g + 4 < TILE_CHUNKS)
            def _():
                issue_loads(g + 4, lax.rem(g + 4, MRING))

            wait_gather(p4, p6)
            compute(p4, p6)
            issue_scatter(p4, p6)
            return carry

        lax.fori_loop(0, TILE_CHUNKS, body, 0)
        for q in range(4):
            g = TILE_CHUNKS - 4 + q
            wait_scatter(g % DRING, g % MRING)

    @pl.when(c == 0)
    def _():
        run_half(ego_l, side_l)

    @pl.when(c == 1)
    def _():
        run_half(ego_r, side_r)

    plsc.subcore_barrier()

    @pl.when(c == 0)
    def _():
        pltpu.sync_copy(acc_sh.at[pl.ds(s * STRIPE, STRIPE)],
                        side_l.at[pl.ds(s * STRIPE, STRIPE)])

    @pl.when(c == 1)
    def _():
        pltpu.sync_copy(acc_sh.at[pl.ds(s * STRIPE, STRIPE)],
                        side_r.at[pl.ds(s * STRIPE, STRIPE)])


@jax.jit
def _sc_spmm(meta_c, vals_c, zeros, ego_l, ego_r):
    mesh = plsc.VectorSubcoreMesh(core_axis_name="c", subcore_axis_name="s")
    f = pl.kernel(
        _spmm_body,
        out_type=(
            jax.ShapeDtypeStruct((N_PAD, HDIM), jnp.float32),
            jax.ShapeDtypeStruct((N_PAD, HDIM), jnp.float32),
        ),
        mesh=mesh,
        scratch_types=[
            pltpu.VMEM((MRING, 2, CHUNK), jnp.int32),
            pltpu.VMEM((MRING, CHUNK), jnp.float32),
            pltpu.VMEM((DRING, CHUNK, HDIM), jnp.float32),
            pltpu.SemaphoreType.DMA((DRING,)),
            pltpu.SemaphoreType.DMA((DRING,)),
            pltpu.SemaphoreType.DMA((MRING,)),
            pltpu.VMEM_SHARED((N_PAD, HDIM), jnp.float32),
        ],
        compiler_params=pltpu.CompilerParams(use_tc_tiling_on_sc=False),
    )
    return f(meta_c, vals_c, zeros, ego_l, ego_r)


N_Q = N_PAD // 4


def _dense_body(sl, sr, el, er, al, ar, wl, wr, b4l, b4r, s32, s32t,
                nl, nr, aol, aor):
    x = jnp.concatenate([sl[...], sr[...]], axis=1)
    e = jnp.concatenate([el[...], er[...]], axis=1)
    xx = jnp.concatenate([x, e * x], axis=1)
    dn = (((1,), (0,)), ((), ()))
    hl = lax.dot_general(xx, wl[...], dn,
                         preferred_element_type=jnp.float32) + b4l[...]
    hr = lax.dot_general(xx, wr[...], dn,
                         preferred_element_type=jnp.float32) + b4r[...]
    gl = jnp.where(hl >= 0, hl, 0.2 * hl)
    gr = jnp.where(hr >= 0, hr, 0.2 * hr)
    n2 = (lax.dot_general(gl * gl, s32[...], dn,
                          preferred_element_type=jnp.float32) +
          lax.dot_general(gr * gr, s32[...], dn,
                          preferred_element_type=jnp.float32))
    inv = 1.0 / jnp.maximum(jnp.sqrt(n2), 1e-12)
    inv128 = lax.dot_general(inv, s32t[...], dn,
                             preferred_element_type=jnp.float32)
    nl[...] = gl
    nr[...] = gr
    aol[...] = al[...] + gl * inv128
    aor[...] = ar[...] + gr * inv128


@jax.jit
def _tc_dense(sl, sr, el, er, al, ar, wl, wr, b4l, b4r, s32, s32t):
    R = 3128
    grid = N_Q // R
    blk = pl.BlockSpec((R, 128), lambda i: (i, 0))
    wspec = pl.BlockSpec((512, 128), lambda i: (0, 0))
    bspec = pl.BlockSpec((1, 128), lambda i: (0, 0))
    s32spec = pl.BlockSpec((128, 4), lambda i: (0, 0))
    s32tspec = pl.BlockSpec((4, 128), lambda i: (0, 0))
    out = jax.ShapeDtypeStruct((N_Q, 128), jnp.float32)
    return pl.pallas_call(
        _dense_body,
        grid=(grid,),
        in_specs=[blk, blk, blk, blk, blk, blk, wspec, wspec, bspec, bspec,
                  s32spec, s32tspec],
        out_specs=[blk, blk, blk, blk],
        out_shape=[out, out, out, out],
    )(sl, sr, el, er, al, ar, wl, wr, b4l, b4r, s32, s32t)


PER_W = B // (NC * NS)


def _lane_perm(v, mask_xor):
    idx = (jnp.arange(16, dtype=jnp.int32) ^ mask_xor)[:, None]
    dn = lax.GatherDimensionNumbers(offset_dims=(),
                                    collapsed_slice_dims=(0,),
                                    start_index_map=(0,))
    return lax.gather(v, idx, dn, slice_sizes=(1,),
                      mode=lax.GatherScatterMode.PROMISE_IN_BOUNDS)


def _final_body(acc_l, acc_r, user_table, item_table, user, item,
                pred, users_ego, items_ego,
                uidx, iidx, aul, aur, ail, air, tu, ti, pred_b):
    c = lax.axis_index("c")
    s = lax.axis_index("s")
    wid = s * NC + c
    base = wid * PER_W
    pltpu.sync_copy(user.at[pl.ds(base, PER_W)], uidx)
    pltpu.sync_copy(item.at[pl.ds(base, PER_W)], iidx)
    pltpu.sync_copy(user_table.at[uidx], tu)
    pltpu.sync_copy(item_table.at[iidx], ti)
    pltpu.sync_copy(acc_l.at[uidx], aul)
    pltpu.sync_copy(acc_r.at[uidx], aur)
    for k in range(PER_W // 16):
        iidx[pl.ds(k * 16, 16)] = iidx[pl.ds(k * 16, 16)] + N_USERS
    pltpu.sync_copy(acc_l.at[iidx], ail)
    pltpu.sync_copy(acc_r.at[iidx], air)
    pltpu.sync_copy(tu, users_ego.at[pl.ds(base, PER_W)])
    pltpu.sync_copy(ti, items_ego.at[pl.ds(base, PER_W)])

    lanes = jnp.arange(16, dtype=jnp.int32)

    def group_body(p16, carry):
        dots = jnp.zeros((16,), jnp.float32)
        for t in range(16):
            p = p16 * 16 + t
            sv = aul[p, pl.ds(0, 16)] * ail[p, pl.ds(0, 16)]
            sv = sv + aul[p, pl.ds(16, 16)] * ail[p, pl.ds(16, 16)]
            sv = sv + aur[p, pl.ds(0, 16)] * air[p, pl.ds(0, 16)]
            sv = sv + aur[p, pl.ds(16, 16)] * air[p, pl.ds(16, 16)]
            for k in (1, 2, 4, 8):
                sv = sv + _lane_perm(sv, k)
            dots = jnp.where(lanes == t, sv, dots)
        pred_b[pl.ds(p16 * 16, 16)] = dots * (1.0 / 16.0)
        return carry

    lax.fori_loop(0, PER_W // 16, group_body, 0)
    pltpu.sync_copy(pred_b, pred.at[pl.ds(base, PER_W)])


@jax.jit
def _sc_final(acc_l, acc_r, user_table, item_table, user, item):
    mesh = plsc.VectorSubcoreMesh(core_axis_name="c", subcore_axis_name="s")
    f = pl.kernel(
        _final_body,
        out_type=(
            jax.ShapeDtypeStruct((B,), jnp.float32),
            jax.ShapeDtypeStruct((B, EDIM), jnp.float32),
            jax.ShapeDtypeStruct((B, EDIM), jnp.float32),
        ),
        mesh=mesh,
        scratch_types=[
            pltpu.VMEM((PER_W,), jnp.int32),
            pltpu.VMEM((PER_W,), jnp.int32),
            pltpu.VMEM((PER_W, HDIM), jnp.float32),
            pltpu.VMEM((PER_W, HDIM), jnp.float32),
            pltpu.VMEM((PER_W, HDIM), jnp.float32),
            pltpu.VMEM((PER_W, HDIM), jnp.float32),
            pltpu.VMEM((PER_W, EDIM), jnp.float32),
            pltpu.VMEM((PER_W, EDIM), jnp.float32),
            pltpu.VMEM((PER_W,), jnp.float32),
        ],
        compiler_params=pltpu.CompilerParams(use_tc_tiling_on_sc=False),
    )
    return f(acc_l, acc_r, user_table, item_table, user, item)


def kernel(user_table, item_table, W_gc_0, b_gc_0, W_bi_0, b_bi_0,
           W_gc_1, b_gc_1, W_bi_1, b_bi_1, W_gc_2, b_gc_2, W_bi_2, b_bi_2,
           vals, rows, cols, user, u_ir, nbr, item, rate):
    gc = [(W_gc_0, b_gc_0), (W_gc_1, b_gc_1), (W_gc_2, b_gc_2)]
    bi = [(W_bi_0, b_bi_0), (W_bi_1, b_bi_1), (W_bi_2, b_bi_2)]

    ego = jnp.concatenate([user_table, item_table], axis=0)
    ego_p = jnp.pad(ego, ((0, N_PAD - N_NODES), (0, 0)))
    ego_l = ego_p[:, :HDIM]
    ego_r = ego_p[:, HDIM:]
    el_p = ego_l.reshape(N_Q, 128)
    er_p = ego_r.reshape(N_Q, 128)
    al_p = el_p
    ar_p = er_p

    epad = NNZ_PAD - NNZ
    cshape = (NS * TILE_CHUNKS, CHUNK)
    cols2 = jnp.pad(cols, (0, epad)).reshape(cshape)
    rows2 = jnp.pad(rows, (0, epad)).reshape(cshape)
    vals2 = jnp.pad(vals, (0, epad)).reshape(cshape)
    meta_c = jnp.stack([cols2, rows2], axis=1)
    zeros = jnp.zeros((N_PAD, HDIM), jnp.float32)

    eye4 = jnp.eye(4, dtype=jnp.float32)
    s32 = jnp.kron(eye4, jnp.ones((HDIM, 1), jnp.float32))
    s32t = jnp.kron(eye4, jnp.ones((1, HDIM), jnp.float32))

    for l in range(3):
        side_l, side_r = _sc_spmm(meta_c, vals2, zeros, ego_l, ego_r)
        a = gc[l][0].T
        b = bi[l][0].T
        wl = jnp.concatenate([
            jnp.kron(eye4, a[:HDIM, :HDIM]),
            jnp.kron(eye4, a[HDIM:, :HDIM]),
            jnp.kron(eye4, b[:HDIM, :HDIM]),
            jnp.kron(eye4, b[HDIM:, :HDIM]),
        ], axis=0)
        wr = jnp.concatenate([
            jnp.kron(eye4, a[:HDIM, HDIM:]),
            jnp.kron(eye4, a[HDIM:, HDIM:]),
            jnp.kron(eye4, b[:HDIM, HDIM:]),
            jnp.kron(eye4, b[HDIM:, HDIM:]),
        ], axis=0)
        bsum = gc[l][1] + bi[l][1]
        b4l = jnp.tile(bsum[:HDIM], 4).reshape(1, 128)
        b4r = jnp.tile(bsum[HDIM:], 4).reshape(1, 128)
        el_p, er_p, al_p, ar_p = _tc_dense(
            side_l.reshape(N_Q, 128), side_r.reshape(N_Q, 128),
            el_p, er_p, al_p, ar_p, wl, wr, b4l, b4r, s32, s32t)
        ego_l = el_p.reshape(N_PAD, HDIM)
        ego_r = er_p.reshape(N_PAD, HDIM)

    pred, users_ego, items_ego = _sc_final(al_p.reshape(N_PAD, HDIM),
                                           ar_p.reshape(N_PAD, HDIM),
                                           user_table, item_table,
                                           user, item)
    return (pred, users_ego, items_ego)


# speedup vs baseline: 9.6850x; 1.0871x over previous
import jax
import jax.numpy as jnp
from jax import lax
from jax.experimental import pallas as pl
from jax.experimental.pallas import tpu as pltpu
from jax.experimental.pallas import tpu_sc as plsc

N_USERS = 25000
N_ITEMS = 25000
N_NODES = N_USERS + N_ITEMS
EDIM = 64
HDIM = 32
NNZ = 800000
B = 4096

NC = 2
NS = 16
CHUNK = 192
TILE_CHUNKS = 264
NNZ_PAD = NS * TILE_CHUNKS * CHUNK

N_PAD = 50048
STRIPE = N_PAD // NS
DRING = 4
MRING = 6


def _spmm_body(cols_c, rows_c, vals_c, zeros, ego_l, ego_r, side_l, side_r,
               cols_b, rows_b, vals_b, data_b, sem_g, sem_s, sem_ld, acc_sh):
    c = lax.axis_index("c")
    s = lax.axis_index("s")

    pltpu.sync_copy(zeros.at[pl.ds(s * STRIPE, STRIPE)],
                    acc_sh.at[pl.ds(s * STRIPE, STRIPE)])
    plsc.subcore_barrier()

    def chunk_of(g):
        return s * TILE_CHUNKS + g

    def issue_loads(g, m):
        e0 = chunk_of(g) * CHUNK
        pltpu.async_copy(cols_c.at[pl.ds(e0, CHUNK)], cols_b.at[m],
                         sem_ld.at[m])
        pltpu.async_copy(rows_c.at[pl.ds(e0, CHUNK)], rows_b.at[m],
                         sem_ld.at[m])
        pltpu.async_copy(vals_c.at[pl.ds(e0, CHUNK)], vals_b.at[m],
                         sem_ld.at[m])

    def wait_loads(g, m):
        e0 = chunk_of(g) * CHUNK
        pltpu.make_async_copy(cols_c.at[pl.ds(e0, CHUNK)], cols_b.at[m],
                              sem_ld.at[m]).wait()
        pltpu.make_async_copy(rows_c.at[pl.ds(e0, CHUNK)], rows_b.at[m],
                              sem_ld.at[m]).wait()
        pltpu.make_async_copy(vals_c.at[pl.ds(e0, CHUNK)], vals_b.at[m],
                              sem_ld.at[m]).wait()

    def run_half(ego_hbm, side_hbm):
        def issue_gather(p, m):
            pltpu.async_copy(ego_hbm.at[cols_b.at[m]], data_b.at[p],
                             sem_g.at[p])

        def wait_gather(p, m):
            pltpu.make_async_copy(ego_hbm.at[cols_b.at[m]],
                                  data_b.at[p], sem_g.at[p]).wait()

        def issue_scatter(p, m):
            pltpu.async_copy(data_b.at[p], acc_sh.at[rows_b.at[m]],
                             sem_s.at[p], add=True)

        def wait_scatter(p, m):
            pltpu.make_async_copy(data_b.at[p], acc_sh.at[rows_b.at[m]],
                                  sem_s.at[p]).wait()

        def compute(p, m):
            def edge_body(e16, carry2):
                vv = vals_b[m, pl.ds(e16 * 16, 16)]
                for t in range(16):
                    e = e16 * 16 + t
                    v = vv[t]
                    d0 = data_b[p, e, pl.ds(0, 16)]
                    data_b[p, e, pl.ds(0, 16)] = d0 * v
                    d1 = data_b[p, e, pl.ds(16, 16)]
                    data_b[p, e, pl.ds(16, 16)] = d1 * v
                return carry2

            lax.fori_loop(0, CHUNK // 16, edge_body, 0)

        for m in range(4):
            issue_loads(m, m)
        wait_loads(0, 0)
        issue_gather(0, 0)
        wait_loads(1, 1)
        issue_gather(1, 1)

        def body(g, carry):
            p4 = lax.rem(g, DRING)
            p6 = lax.rem(g, MRING)
            p4n = lax.rem(g + 2, DRING)
            p6n = lax.rem(g + 2, MRING)

            @pl.when(jnp.logical_and(g >= 2, g + 2 < TILE_CHUNKS))
            def _():
                wait_scatter(p4n, p6n)

            @pl.when(g + 2 < TILE_CHUNKS)
            def _():
                wait_loads(g + 2, p6n)
                issue_gather(p4n, p6n)

            @pl.when(g + 4 < TILE_CHUNKS)
            def _():
                issue_loads(g + 4, lax.rem(g + 4, MRING))

            wait_gather(p4, p6)
            compute(p4, p6)
            issue_scatter(p4, p6)
            return carry

        lax.fori_loop(0, TILE_CHUNKS, body, 0)
        for q in range(4):
            g = TILE_CHUNKS - 4 + q
            wait_scatter(g % DRING, g % MRING)

    @pl.when(c == 0)
    def _():
        run_half(ego_l, side_l)

    @pl.when(c == 1)
    def _():
        run_half(ego_r, side_r)

    plsc.subcore_barrier()

    @pl.when(c == 0)
    def _():
        pltpu.sync_copy(acc_sh.at[pl.ds(s * STRIPE, STRIPE)],
                        side_l.at[pl.ds(s * STRIPE, STRIPE)])

    @pl.when(c == 1)
    def _():
        pltpu.sync_copy(acc_sh.at[pl.ds(s * STRIPE, STRIPE)],
                        side_r.at[pl.ds(s * STRIPE, STRIPE)])


@jax.jit
def _sc_spmm(cols_c, rows_c, vals_c, zeros, ego_l, ego_r):
    mesh = plsc.VectorSubcoreMesh(core_axis_name="c", subcore_axis_name="s")
    f = pl.kernel(
        _spmm_body,
        out_type=(
            jax.ShapeDtypeStruct((N_PAD, HDIM), jnp.float32),
            jax.ShapeDtypeStruct((N_PAD, HDIM), jnp.float32),
        ),
        mesh=mesh,
        scratch_types=[
            pltpu.VMEM((MRING, CHUNK), jnp.int32),
            pltpu.VMEM((MRING, CHUNK), jnp.int32),
            pltpu.VMEM((MRING, CHUNK), jnp.float32),
            pltpu.VMEM((DRING, CHUNK, HDIM), jnp.float32),
            pltpu.SemaphoreType.DMA((DRING,)),
            pltpu.SemaphoreType.DMA((DRING,)),
            pltpu.SemaphoreType.DMA((MRING,)),
            pltpu.VMEM_SHARED((N_PAD, HDIM), jnp.float32),
        ],
        compiler_params=pltpu.CompilerParams(use_tc_tiling_on_sc=False),
    )
    return f(cols_c, rows_c, vals_c, zeros, ego_l, ego_r)


N_Q = N_PAD // 4


def _dense_body(sl, sr, el, er, al, ar, wl, wr, b4l, b4r, s32, s32t,
                nl, nr, aol, aor):
    x = jnp.concatenate([sl[...], sr[...]], axis=1)
    e = jnp.concatenate([el[...], er[...]], axis=1)
    xx = jnp.concatenate([x, e * x], axis=1)
    dn = (((1,), (0,)), ((), ()))
    hl = lax.dot_general(xx, wl[...], dn,
                         preferred_element_type=jnp.float32) + b4l[...]
    hr = lax.dot_general(xx, wr[...], dn,
                         preferred_element_type=jnp.float32) + b4r[...]
    gl = jnp.where(hl >= 0, hl, 0.2 * hl)
    gr = jnp.where(hr >= 0, hr, 0.2 * hr)
    n2 = (lax.dot_general(gl * gl, s32[...], dn,
                          preferred_element_type=jnp.float32) +
          lax.dot_general(gr * gr, s32[...], dn,
                          preferred_element_type=jnp.float32))
    inv = 1.0 / jnp.maximum(jnp.sqrt(n2), 1e-12)
    inv128 = lax.dot_general(inv, s32t[...], dn,
                             preferred_element_type=jnp.float32)
    nl[...] = gl
    nr[...] = gr
    aol[...] = al[...] + gl * inv128
    aor[...] = ar[...] + gr * inv128


@jax.jit
def _tc_dense(sl, sr, el, er, al, ar, wl, wr, b4l, b4r, s32, s32t):
    R = 3128
    grid = N_Q // R
    blk = pl.BlockSpec((R, 128), lambda i: (i, 0))
    wspec = pl.BlockSpec((512, 128), lambda i: (0, 0))
    bspec = pl.BlockSpec((1, 128), lambda i: (0, 0))
    s32spec = pl.BlockSpec((128, 4), lambda i: (0, 0))
    s32tspec = pl.BlockSpec((4, 128), lambda i: (0, 0))
    out = jax.ShapeDtypeStruct((N_Q, 128), jnp.float32)
    return pl.pallas_call(
        _dense_body,
        grid=(grid,),
        in_specs=[blk, blk, blk, blk, blk, blk, wspec, wspec, bspec, bspec,
                  s32spec, s32tspec],
        out_specs=[blk, blk, blk, blk],
        out_shape=[out, out, out, out],
    )(sl, sr, el, er, al, ar, wl, wr, b4l, b4r, s32, s32t)


PER_W = B // (NC * NS)


def _lane_perm(v, mask_xor):
    idx = (jnp.arange(16, dtype=jnp.int32) ^ mask_xor)[:, None]
    dn = lax.GatherDimensionNumbers(offset_dims=(),
                                    collapsed_slice_dims=(0,),
                                    start_index_map=(0,))
    return lax.gather(v, idx, dn, slice_sizes=(1,),
                      mode=lax.GatherScatterMode.PROMISE_IN_BOUNDS)


def _final_body(acc_l, acc_r, user_table, item_table, user, item,
                pred, users_ego, items_ego,
                uidx, iidx, aul, aur, ail, air, tu, ti, pred_b):
    c = lax.axis_index("c")
    s = lax.axis_index("s")
    wid = s * NC + c
    base = wid * PER_W
    pltpu.sync_copy(user.at[pl.ds(base, PER_W)], uidx)
    pltpu.sync_copy(item.at[pl.ds(base, PER_W)], iidx)
    pltpu.sync_copy(user_table.at[uidx], tu)
    pltpu.sync_copy(item_table.at[iidx], ti)
    pltpu.sync_copy(acc_l.at[uidx], aul)
    pltpu.sync_copy(acc_r.at[uidx], aur)
    for k in range(PER_W // 16):
        iidx[pl.ds(k * 16, 16)] = iidx[pl.ds(k * 16, 16)] + N_USERS
    pltpu.sync_copy(acc_l.at[iidx], ail)
    pltpu.sync_copy(acc_r.at[iidx], air)
    pltpu.sync_copy(tu, users_ego.at[pl.ds(base, PER_W)])
    pltpu.sync_copy(ti, items_ego.at[pl.ds(base, PER_W)])

    lanes = jnp.arange(16, dtype=jnp.int32)

    def group_body(p16, carry):
        dots = jnp.zeros((16,), jnp.float32)
        for t in range(16):
            p = p16 * 16 + t
            sv = aul[p, pl.ds(0, 16)] * ail[p, pl.ds(0, 16)]
            sv = sv + aul[p, pl.ds(16, 16)] * ail[p, pl.ds(16, 16)]
            sv = sv + aur[p, pl.ds(0, 16)] * air[p, pl.ds(0, 16)]
            sv = sv + aur[p, pl.ds(16, 16)] * air[p, pl.ds(16, 16)]
            for k in (1, 2, 4, 8):
                sv = sv + _lane_perm(sv, k)
            dots = jnp.where(lanes == t, sv, dots)
        pred_b[pl.ds(p16 * 16, 16)] = dots * (1.0 / 16.0)
        return carry

    lax.fori_loop(0, PER_W // 16, group_body, 0)
    pltpu.sync_copy(pred_b, pred.at[pl.ds(base, PER_W)])


@jax.jit
def _sc_final(acc_l, acc_r, user_table, item_table, user, item):
    mesh = plsc.VectorSubcoreMesh(core_axis_name="c", subcore_axis_name="s")
    f = pl.kernel(
        _final_body,
        out_type=(
            jax.ShapeDtypeStruct((B,), jnp.float32),
            jax.ShapeDtypeStruct((B, EDIM), jnp.float32),
            jax.ShapeDtypeStruct((B, EDIM), jnp.float32),
        ),
        mesh=mesh,
        scratch_types=[
            pltpu.VMEM((PER_W,), jnp.int32),
            pltpu.VMEM((PER_W,), jnp.int32),
            pltpu.VMEM((PER_W, HDIM), jnp.float32),
            pltpu.VMEM((PER_W, HDIM), jnp.float32),
            pltpu.VMEM((PER_W, HDIM), jnp.float32),
            pltpu.VMEM((PER_W, HDIM), jnp.float32),
            pltpu.VMEM((PER_W, EDIM), jnp.float32),
            pltpu.VMEM((PER_W, EDIM), jnp.float32),
            pltpu.VMEM((PER_W,), jnp.float32),
        ],
        compiler_params=pltpu.CompilerParams(use_tc_tiling_on_sc=False),
    )
    return f(acc_l, acc_r, user_table, item_table, user, item)


def kernel(user_table, item_table, W_gc_0, b_gc_0, W_bi_0, b_bi_0,
           W_gc_1, b_gc_1, W_bi_1, b_bi_1, W_gc_2, b_gc_2, W_bi_2, b_bi_2,
           vals, rows, cols, user, u_ir, nbr, item, rate):
    gc = [(W_gc_0, b_gc_0), (W_gc_1, b_gc_1), (W_gc_2, b_gc_2)]
    bi = [(W_bi_0, b_bi_0), (W_bi_1, b_bi_1), (W_bi_2, b_bi_2)]

    ego = jnp.concatenate([user_table, item_table], axis=0)
    ego_p = jnp.pad(ego, ((0, N_PAD - N_NODES), (0, 0)))
    ego_l = ego_p[:, :HDIM]
    ego_r = ego_p[:, HDIM:]
    el_p = ego_l.reshape(N_Q, 128)
    er_p = ego_r.reshape(N_Q, 128)
    al_p = el_p
    ar_p = er_p

    epad = NNZ_PAD - NNZ
    cols1 = jnp.pad(cols, (0, epad))
    rows1 = jnp.pad(rows, (0, epad))
    vals1 = jnp.pad(vals, (0, epad))
    zeros = jnp.zeros((N_PAD, HDIM), jnp.float32)

    eye4 = jnp.eye(4, dtype=jnp.float32)
    s32 = jnp.kron(eye4, jnp.ones((HDIM, 1), jnp.float32))
    s32t = jnp.kron(eye4, jnp.ones((1, HDIM), jnp.float32))

    for l in range(3):
        side_l, side_r = _sc_spmm(cols1, rows1, vals1, zeros, ego_l, ego_r)
        a = gc[l][0].T
        b = bi[l][0].T
        wl = jnp.concatenate([
            jnp.kron(eye4, a[:HDIM, :HDIM]),
            jnp.kron(eye4, a[HDIM:, :HDIM]),
            jnp.kron(eye4, b[:HDIM, :HDIM]),
            jnp.kron(eye4, b[HDIM:, :HDIM]),
        ], axis=0)
        wr = jnp.concatenate([
            jnp.kron(eye4, a[:HDIM, HDIM:]),
            jnp.kron(eye4, a[HDIM:, HDIM:]),
            jnp.kron(eye4, b[:HDIM, HDIM:]),
            jnp.kron(eye4, b[HDIM:, HDIM:]),
        ], axis=0)
        bsum = gc[l][1] + bi[l][1]
        b4l = jnp.tile(bsum[:HDIM], 4).reshape(1, 128)
        b4r = jnp.tile(bsum[HDIM:], 4).reshape(1, 128)
        el_p, er_p, al_p, ar_p = _tc_dense(
            side_l.reshape(N_Q, 128), side_r.reshape(N_Q, 128),
            el_p, er_p, al_p, ar_p, wl, wr, b4l, b4r, s32, s32t)
        ego_l = el_p.reshape(N_PAD, HDIM)
        ego_r = er_p.reshape(N_PAD, HDIM)

    pred, users_ego, items_ego = _sc_final(al_p.reshape(N_PAD, HDIM),
                                           ar_p.reshape(N_PAD, HDIM),
                                           user_table, item_table,
                                           user, item)
    return (pred, users_ego, items_ego)


# speedup vs baseline: 9.7328x; 1.0049x over previous
import jax
import jax.numpy as jnp
from jax import lax
from jax.experimental import pallas as pl
from jax.experimental.pallas import tpu as pltpu
from jax.experimental.pallas import tpu_sc as plsc

N_USERS = 25000
N_ITEMS = 25000
N_NODES = N_USERS + N_ITEMS
EDIM = 64
HDIM = 32
NNZ = 800000
B = 4096

NC = 2
NS = 16
CHUNK = 192
TILE_CHUNKS = 264
NNZ_PAD = NS * TILE_CHUNKS * CHUNK

N_PAD = 50048
STRIPE = N_PAD // NS
DRING = 4
MRING = 6


def _spmm_body(cols_c, rows_c, vals_c, zeros, ego_l, ego_r, side_l, side_r,
               cols_b, rows_b, vals_b, data_b, sem_g, sem_s, sem_ld, acc_sh):
    c = lax.axis_index("c")
    s = lax.axis_index("s")

    pltpu.sync_copy(zeros.at[pl.ds(s * STRIPE, STRIPE)],
                    acc_sh.at[pl.ds(s * STRIPE, STRIPE)])
    plsc.subcore_barrier()

    def chunk_of(g):
        return s * TILE_CHUNKS + g

    def issue_loads(g, m):
        e0 = chunk_of(g) * CHUNK
        pltpu.async_copy(cols_c.at[pl.ds(e0, CHUNK)], cols_b.at[m],
                         sem_ld.at[m])
        pltpu.async_copy(rows_c.at[pl.ds(e0, CHUNK)], rows_b.at[m],
                         sem_ld.at[m])
        pltpu.async_copy(vals_c.at[pl.ds(e0, CHUNK)], vals_b.at[m],
                         sem_ld.at[m])

    def wait_loads(g, m):
        e0 = chunk_of(g) * CHUNK
        pltpu.make_async_copy(cols_c.at[pl.ds(e0, CHUNK)], cols_b.at[m],
                              sem_ld.at[m]).wait()
        pltpu.make_async_copy(rows_c.at[pl.ds(e0, CHUNK)], rows_b.at[m],
                              sem_ld.at[m]).wait()
        pltpu.make_async_copy(vals_c.at[pl.ds(e0, CHUNK)], vals_b.at[m],
                              sem_ld.at[m]).wait()

    def run_half(ego_hbm, side_hbm):
        def issue_gather(p, m):
            pltpu.async_copy(ego_hbm.at[cols_b.at[m]], data_b.at[p],
                             sem_g.at[p])

        def wait_gather(p, m):
            pltpu.make_async_copy(ego_hbm.at[cols_b.at[m]],
                                  data_b.at[p], sem_g.at[p]).wait()

        def issue_scatter(p, m):
            pltpu.async_copy(data_b.at[p], acc_sh.at[rows_b.at[m]],
                             sem_s.at[p], add=True)

        def wait_scatter(p, m):
            pltpu.make_async_copy(data_b.at[p], acc_sh.at[rows_b.at[m]],
                                  sem_s.at[p]).wait()

        def compute(p, m):
            def edge_body(e16, carry2):
                vv = vals_b[m, pl.ds(e16 * 16, 16)]
                for t in range(16):
                    e = e16 * 16 + t
                    v = vv[t]
                    d0 = data_b[p, e, pl.ds(0, 16)]
                    data_b[p, e, pl.ds(0, 16)] = d0 * v
                    d1 = data_b[p, e, pl.ds(16, 16)]
                    data_b[p, e, pl.ds(16, 16)] = d1 * v
                return carry2

            lax.fori_loop(0, CHUNK // 16, edge_body, 0)

        for m in range(4):
            issue_loads(m, m)
        wait_loads(0, 0)
        issue_gather(0, 0)
        wait_loads(1, 1)
        issue_gather(1, 1)

        def body(g, carry):
            p4 = lax.rem(g, DRING)
            p6 = lax.rem(g, MRING)
            p4n = lax.rem(g + 2, DRING)
            p6n = lax.rem(g + 2, MRING)

            @pl.when(jnp.logical_and(g >= 2, g + 2 < TILE_CHUNKS))
            def _():
                wait_scatter(p4n, p6n)

            @pl.when(g + 2 < TILE_CHUNKS)
            def _():
                wait_loads(g + 2, p6n)
                issue_gather(p4n, p6n)

            @pl.when(g + 4 < TILE_CHUNKS)
            def _():
                issue_loads(g + 4, lax.rem(g + 4, MRING))

            wait_gather(p4, p6)
            compute(p4, p6)
            issue_scatter(p4, p6)
            return carry

        lax.fori_loop(0, TILE_CHUNKS, body, 0)
        for q in range(4):
            g = TILE_CHUNKS - 4 + q
            wait_scatter(g % DRING, g % MRING)

    @pl.when(c == 0)
    def _():
        run_half(ego_l, side_l)

    @pl.when(c == 1)
    def _():
        run_half(ego_r, side_r)

    plsc.subcore_barrier()

    @pl.when(c == 0)
    def _():
        pltpu.sync_copy(acc_sh.at[pl.ds(s * STRIPE, STRIPE)],
                        side_l.at[pl.ds(s * STRIPE, STRIPE)])

    @pl.when(c == 1)
    def _():
        pltpu.sync_copy(acc_sh.at[pl.ds(s * STRIPE, STRIPE)],
                        side_r.at[pl.ds(s * STRIPE, STRIPE)])


@jax.jit
def _sc_spmm(cols_c, rows_c, vals_c, zeros, ego_l, ego_r):
    mesh = plsc.VectorSubcoreMesh(core_axis_name="c", subcore_axis_name="s")
    f = pl.kernel(
        _spmm_body,
        out_type=(
            jax.ShapeDtypeStruct((N_PAD, HDIM), jnp.float32),
            jax.ShapeDtypeStruct((N_PAD, HDIM), jnp.float32),
        ),
        mesh=mesh,
        scratch_types=[
            pltpu.VMEM((MRING, CHUNK), jnp.int32),
            pltpu.VMEM((MRING, CHUNK), jnp.int32),
            pltpu.VMEM((MRING, CHUNK), jnp.float32),
            pltpu.VMEM((DRING, CHUNK, HDIM), jnp.float32),
            pltpu.SemaphoreType.DMA((DRING,)),
            pltpu.SemaphoreType.DMA((DRING,)),
            pltpu.SemaphoreType.DMA((MRING,)),
            pltpu.VMEM_SHARED((N_PAD, HDIM), jnp.float32),
        ],
        compiler_params=pltpu.CompilerParams(use_tc_tiling_on_sc=False),
    )
    return f(cols_c, rows_c, vals_c, zeros, ego_l, ego_r)


N_Q = N_PAD // 4


def _dense_body(sl, sr, el, er, al, ar, wl, wr, b4l, b4r, s32, s32t,
                nl, nr, aol, aor):
    x = jnp.concatenate([sl[...], sr[...]], axis=1)
    e = jnp.concatenate([el[...], er[...]], axis=1)
    xx = jnp.concatenate([x, e * x], axis=1)
    dn = (((1,), (0,)), ((), ()))
    hl = lax.dot_general(xx, wl[...], dn,
                         preferred_element_type=jnp.float32) + b4l[...]
    hr = lax.dot_general(xx, wr[...], dn,
                         preferred_element_type=jnp.float32) + b4r[...]
    gl = jnp.where(hl >= 0, hl, 0.2 * hl)
    gr = jnp.where(hr >= 0, hr, 0.2 * hr)
    n2 = (lax.dot_general(gl * gl, s32[...], dn,
                          preferred_element_type=jnp.float32) +
          lax.dot_general(gr * gr, s32[...], dn,
                          preferred_element_type=jnp.float32))
    inv = 1.0 / jnp.maximum(jnp.sqrt(n2), 1e-12)
    inv128 = lax.dot_general(inv, s32t[...], dn,
                             preferred_element_type=jnp.float32)
    nl[...] = gl
    nr[...] = gr
    aol[...] = al[...] + gl * inv128
    aor[...] = ar[...] + gr * inv128


@jax.jit
def _tc_dense(sl, sr, el, er, al, ar, wl, wr, b4l, b4r, s32, s32t):
    R = 3128
    grid = N_Q // R
    blk = pl.BlockSpec((R, 128), lambda i: (i, 0))
    wspec = pl.BlockSpec((512, 128), lambda i: (0, 0))
    bspec = pl.BlockSpec((1, 128), lambda i: (0, 0))
    s32spec = pl.BlockSpec((128, 4), lambda i: (0, 0))
    s32tspec = pl.BlockSpec((4, 128), lambda i: (0, 0))
    out = jax.ShapeDtypeStruct((N_Q, 128), jnp.float32)
    return pl.pallas_call(
        _dense_body,
        grid=(grid,),
        in_specs=[blk, blk, blk, blk, blk, blk, wspec, wspec, bspec, bspec,
                  s32spec, s32tspec],
        out_specs=[blk, blk, blk, blk],
        out_shape=[out, out, out, out],
    )(sl, sr, el, er, al, ar, wl, wr, b4l, b4r, s32, s32t)


PER_W = B // (NC * NS)


def _lane_perm(v, mask_xor):
    idx = (jnp.arange(16, dtype=jnp.int32) ^ mask_xor)[:, None]
    dn = lax.GatherDimensionNumbers(offset_dims=(),
                                    collapsed_slice_dims=(0,),
                                    start_index_map=(0,))
    return lax.gather(v, idx, dn, slice_sizes=(1,),
                      mode=lax.GatherScatterMode.PROMISE_IN_BOUNDS)


def _final_body(acc_l, acc_r, ego0_l, ego0_r, user, item,
                pred, ul_out, ur_out, il_out, ir_out,
                uidx, iidx, aul, aur, ail, air, tul, tur, til, tir, pred_b):
    c = lax.axis_index("c")
    s = lax.axis_index("s")
    wid = s * NC + c
    base = wid * PER_W
    pltpu.sync_copy(user.at[pl.ds(base, PER_W)], uidx)
    pltpu.sync_copy(item.at[pl.ds(base, PER_W)], iidx)
    pltpu.sync_copy(ego0_l.at[uidx], tul)
    pltpu.sync_copy(ego0_r.at[uidx], tur)
    pltpu.sync_copy(acc_l.at[uidx], aul)
    pltpu.sync_copy(acc_r.at[uidx], aur)
    for k in range(PER_W // 16):
        iidx[pl.ds(k * 16, 16)] = iidx[pl.ds(k * 16, 16)] + N_USERS
    pltpu.sync_copy(ego0_l.at[iidx], til)
    pltpu.sync_copy(ego0_r.at[iidx], tir)
    pltpu.sync_copy(acc_l.at[iidx], ail)
    pltpu.sync_copy(acc_r.at[iidx], air)
    pltpu.sync_copy(tul, ul_out.at[pl.ds(base, PER_W)])
    pltpu.sync_copy(tur, ur_out.at[pl.ds(base, PER_W)])
    pltpu.sync_copy(til, il_out.at[pl.ds(base, PER_W)])
    pltpu.sync_copy(tir, ir_out.at[pl.ds(base, PER_W)])

    lanes = jnp.arange(16, dtype=jnp.int32)

    def group_body(p16, carry):
        dots = jnp.zeros((16,), jnp.float32)
        for t in range(16):
            p = p16 * 16 + t
            sv = aul[p, pl.ds(0, 16)] * ail[p, pl.ds(0, 16)]
            sv = sv + aul[p, pl.ds(16, 16)] * ail[p, pl.ds(16, 16)]
            sv = sv + aur[p, pl.ds(0, 16)] * air[p, pl.ds(0, 16)]
            sv = sv + aur[p, pl.ds(16, 16)] * air[p, pl.ds(16, 16)]
            for k in (1, 2, 4, 8):
                sv = sv + _lane_perm(sv, k)
            dots = jnp.where(lanes == t, sv, dots)
        pred_b[pl.ds(p16 * 16, 16)] = dots * (1.0 / 16.0)
        return carry

    lax.fori_loop(0, PER_W // 16, group_body, 0)
    pltpu.sync_copy(pred_b, pred.at[pl.ds(base, PER_W)])


@jax.jit
def _sc_final(acc_l, acc_r, ego0_l, ego0_r, user, item):
    mesh = plsc.VectorSubcoreMesh(core_axis_name="c", subcore_axis_name="s")
    hbuf = pltpu.VMEM((PER_W, HDIM), jnp.float32)
    f = pl.kernel(
        _final_body,
        out_type=(
            jax.ShapeDtypeStruct((B,), jnp.float32),
            jax.ShapeDtypeStruct((B, HDIM), jnp.float32),
            jax.ShapeDtypeStruct((B, HDIM), jnp.float32),
            jax.ShapeDtypeStruct((B, HDIM), jnp.float32),
            jax.ShapeDtypeStruct((B, HDIM), jnp.float32),
        ),
        mesh=mesh,
        scratch_types=[
            pltpu.VMEM((PER_W,), jnp.int32),
            pltpu.VMEM((PER_W,), jnp.int32),
            hbuf, hbuf, hbuf, hbuf, hbuf, hbuf, hbuf, hbuf,
            pltpu.VMEM((PER_W,), jnp.float32),
        ],
        compiler_params=pltpu.CompilerParams(use_tc_tiling_on_sc=False),
    )
    return f(acc_l, acc_r, ego0_l, ego0_r, user, item)


def kernel(user_table, item_table, W_gc_0, b_gc_0, W_bi_0, b_bi_0,
           W_gc_1, b_gc_1, W_bi_1, b_bi_1, W_gc_2, b_gc_2, W_bi_2, b_bi_2,
           vals, rows, cols, user, u_ir, nbr, item, rate):
    gc = [(W_gc_0, b_gc_0), (W_gc_1, b_gc_1), (W_gc_2, b_gc_2)]
    bi = [(W_bi_0, b_bi_0), (W_bi_1, b_bi_1), (W_bi_2, b_bi_2)]

    ego = jnp.concatenate([user_table, item_table], axis=0)
    ego_p = jnp.pad(ego, ((0, N_PAD - N_NODES), (0, 0)))
    ego_l = ego0_l = ego_p[:, :HDIM]
    ego_r = ego0_r = ego_p[:, HDIM:]
    el_p = ego_l.reshape(N_Q, 128)
    er_p = ego_r.reshape(N_Q, 128)
    al_p = el_p
    ar_p = er_p

    epad = NNZ_PAD - NNZ
    cols1 = jnp.pad(cols, (0, epad))
    rows1 = jnp.pad(rows, (0, epad))
    vals1 = jnp.pad(vals, (0, epad))
    zeros = jnp.zeros((N_PAD, HDIM), jnp.float32)

    eye4 = jnp.eye(4, dtype=jnp.float32)
    s32 = jnp.kron(eye4, jnp.ones((HDIM, 1), jnp.float32))
    s32t = jnp.kron(eye4, jnp.ones((1, HDIM), jnp.float32))

    for l in range(3):
        side_l, side_r = _sc_spmm(cols1, rows1, vals1, zeros, ego_l, ego_r)
        a = gc[l][0].T
        b = bi[l][0].T
        wl = jnp.concatenate([
            jnp.kron(eye4, a[:HDIM, :HDIM]),
            jnp.kron(eye4, a[HDIM:, :HDIM]),
            jnp.kron(eye4, b[:HDIM, :HDIM]),
            jnp.kron(eye4, b[HDIM:, :HDIM]),
        ], axis=0)
        wr = jnp.concatenate([
            jnp.kron(eye4, a[:HDIM, HDIM:]),
            jnp.kron(eye4, a[HDIM:, HDIM:]),
            jnp.kron(eye4, b[:HDIM, HDIM:]),
            jnp.kron(eye4, b[HDIM:, HDIM:]),
        ], axis=0)
        bsum = gc[l][1] + bi[l][1]
        b4l = jnp.tile(bsum[:HDIM], 4).reshape(1, 128)
        b4r = jnp.tile(bsum[HDIM:], 4).reshape(1, 128)
        el_p, er_p, al_p, ar_p = _tc_dense(
            side_l.reshape(N_Q, 128), side_r.reshape(N_Q, 128),
            el_p, er_p, al_p, ar_p, wl, wr, b4l, b4r, s32, s32t)
        ego_l = el_p.reshape(N_PAD, HDIM)
        ego_r = er_p.reshape(N_PAD, HDIM)

    pred, ul, ur, il, ir = _sc_final(al_p.reshape(N_PAD, HDIM),
                                     ar_p.reshape(N_PAD, HDIM),
                                     ego0_l, ego0_r, user, item)
    users_ego = jnp.concatenate([ul, ur], axis=1)
    items_ego = jnp.concatenate([il, ir], axis=1)
    return (pred, users_ego, items_ego)
